# baseline (speedup 1.0000x reference)
import functools

import jax
import jax.numpy as jnp
from jax.experimental import pallas as pl
from jax.experimental.pallas import tpu as pltpu


def _rup(x, m):
    return (x + m - 1) // m * m


def _rs_of(H, W):
    base = (H + 2) * (W + 2)
    slack = 2 * (W + 2) + 2
    return _rup(base, 16) + slack


def _mmk(a_ref, b_ref, s_ref, t_ref, o_ref, *, relu):
    acc = jnp.dot(a_ref[...], b_ref[...], preferred_element_type=jnp.float32)
    y = acc * s_ref[...] + t_ref[...]
    if relu:
        y = jnp.maximum(y, 0.0)
    o_ref[...] = y.astype(o_ref.dtype)


def _matmul_bn(a, b, scale, shift, *, relu, tm=1024):
    M, K = a.shape
    N = b.shape[1]
    tm = min(tm, _rup(M, 8))
    Mp = _rup(M, tm)
    a = jnp.pad(a, ((0, Mp - M), (0, 0)))
    out = pl.pallas_call(
        functools.partial(_mmk, relu=relu),
        out_shape=jax.ShapeDtypeStruct((Mp, N), jnp.bfloat16),
        grid=(Mp // tm,),
        in_specs=[
            pl.BlockSpec((tm, K), lambda i: (i, 0)),
            pl.BlockSpec((K, N), lambda i: (0, 0)),
            pl.BlockSpec((1, N), lambda i: (0, 0)),
            pl.BlockSpec((1, N), lambda i: (0, 0)),
        ],
        out_specs=pl.BlockSpec((tm, N), lambda i: (i, 0)),
        compiler_params=pltpu.CompilerParams(
            dimension_semantics=("parallel",),
            vmem_limit_bytes=100 * 1024 * 1024),
    )(a, b.astype(jnp.bfloat16), scale.reshape(1, N).astype(jnp.float32),
      shift.reshape(1, N).astype(jnp.float32))
    return out[:M]


def _col_and_acc(x_refs, w_ref, *, M, Wp):
    pieces = []
    for x_ref in x_refs:
        xv = x_ref[0, 0]
        for dh in range(3):
            for dw in range(3):
                off = dh * Wp + dw
                pieces.append(xv[off:off + M, :])
    col = jnp.concatenate(pieces, axis=-1)
    return jnp.dot(col, w_ref[...], preferred_element_type=jnp.float32)


def _interior_mask(shape, *, H, W):
    Wp = W + 2
    r = jax.lax.broadcasted_iota(jnp.int32, shape, 0) + (Wp + 1)
    h = jnp.floor((r.astype(jnp.float32) + 0.5) *
                  jnp.float32(1.0 / Wp)).astype(jnp.int32)
    w = r - h * Wp
    return (h >= 1) & (h <= H) & (w >= 1) & (w <= W)


def _conv_s1_kernel(*refs, H, W, has_res, relu, Tp):
    if has_res:
        x0_ref, x1_ref, x2_ref, w_ref, s_ref, t_ref, res_ref, o_ref = refs
    else:
        x0_ref, x1_ref, x2_ref, w_ref, s_ref, t_ref, o_ref = refs
    tp = pl.program_id(1)
    Wp = W + 2
    RS = o_ref.shape[2]
    M = RS - (2 * Wp + 2)

    @pl.when(jnp.logical_or(tp == 0, tp == Tp - 1))
    def _():
        o_ref[...] = jnp.zeros_like(o_ref)

    @pl.when(jnp.logical_and(tp > 0, tp < Tp - 1))
    def _():
        D = Wp + 1
        acc = _col_and_acc((x0_ref, x1_ref, x2_ref), w_ref, M=M, Wp=Wp)
        y = acc * s_ref[...] + t_ref[...]
        if has_res:
            y = y + res_ref[0, 0, D:D + M, :].astype(jnp.float32)
        if relu:
            y = jnp.maximum(y, 0.0)
        y = jnp.where(_interior_mask(y.shape, H=H, W=W), y, 0.0)
        C = y.shape[1]
        o_ref[0, 0, 0:D, :] = jnp.zeros((D, C), o_ref.dtype)
        o_ref[0, 0, D:D + M, :] = y.astype(o_ref.dtype)
        o_ref[0, 0, D + M:RS, :] = jnp.zeros((RS - D - M, C), o_ref.dtype)


def _conv_s1(x, w, scale, shift, *, H, W, residual=None, relu=True):
    N, Tp, RS, Cin = x.shape
    Cout = w.shape[0]
    wm = jnp.transpose(w, (2, 3, 4, 1, 0)).reshape(27 * Cin, Cout)
    wm = wm.astype(jnp.bfloat16)
    sc = scale.reshape(1, Cout).astype(jnp.float32)
    sh = shift.reshape(1, Cout).astype(jnp.float32)
    in_specs = [
        pl.BlockSpec((1, 1, RS, Cin),
                     lambda n, t: (n, jnp.maximum(t - 1, 0), 0, 0)),
        pl.BlockSpec((1, 1, RS, Cin), lambda n, t: (n, t, 0, 0)),
        pl.BlockSpec((1, 1, RS, Cin),
                     lambda n, t: (n, jnp.minimum(t + 1, Tp - 1), 0, 0)),
        pl.BlockSpec((27 * Cin, Cout), lambda n, t: (0, 0)),
        pl.BlockSpec((1, Cout), lambda n, t: (0, 0)),
        pl.BlockSpec((1, Cout), lambda n, t: (0, 0)),
    ]
    args = [x, x, x, wm, sc, sh]
    if residual is not None:
        in_specs.append(pl.BlockSpec((1, 1, RS, Cout),
                                     lambda n, t: (n, t, 0, 0)))
        args.append(residual)
    kern = functools.partial(_conv_s1_kernel, H=H, W=W,
                             has_res=residual is not None, relu=relu, Tp=Tp)
    return pl.pallas_call(
        kern,
        out_shape=jax.ShapeDtypeStruct((N, Tp, RS, Cout), jnp.bfloat16),
        grid=(N, Tp),
        in_specs=in_specs,
        out_specs=pl.BlockSpec((1, 1, RS, Cout), lambda n, t: (n, t, 0, 0)),
        compiler_params=pltpu.CompilerParams(
            dimension_semantics=("parallel", "arbitrary"),
            vmem_limit_bytes=100 * 1024 * 1024),
    )(*args)


def _conv_pool_fc_kernel(x0_ref, x1_ref, x2_ref, w_ref, s_ref, t_ref,
                         res_ref, fcw_ref, fcb_ref, o_ref, *, H, W):
    Wp = W + 2
    RS = x1_ref.shape[2]
    M = RS - (2 * Wp + 2)
    D = Wp + 1
    acc = _col_and_acc((x0_ref, x1_ref, x2_ref), w_ref, M=M, Wp=Wp)
    y = acc * s_ref[...] + t_ref[...]
    y = y + res_ref[0, 0, D:D + M, :].astype(jnp.float32)
    y = jnp.maximum(y, 0.0)
    y = jnp.where(_interior_mask(y.shape, H=H, W=W), y, 0.0)
    pooled = jnp.sum(y, axis=0, keepdims=True) * (1.0 / (H * W))
    o_ref[0] = (jnp.dot(pooled, fcw_ref[...],
                        preferred_element_type=jnp.float32) + fcb_ref[...])


def _conv_pool_fc(x, w, scale, shift, residual, fc_w, fc_b, *, H, W):
    N, Tp, RS, Cin = x.shape
    Cout = w.shape[0]
    nc = fc_w.shape[1]
    NCp = _rup(nc, 128)
    wm = jnp.transpose(w, (2, 3, 4, 1, 0)).reshape(27 * Cin, Cout)
    wm = wm.astype(jnp.bfloat16)
    fcw = jnp.pad(fc_w.astype(jnp.float32), ((0, 0), (0, NCp - nc)))
    fcb = jnp.pad(fc_b.astype(jnp.float32), (0, NCp - nc)).reshape(1, NCp)
    out = pl.pallas_call(
        functools.partial(_conv_pool_fc_kernel, H=H, W=W),
        out_shape=jax.ShapeDtypeStruct((N, 1, NCp), jnp.float32),
        grid=(N,),
        in_specs=[
            pl.BlockSpec((1, 1, RS, Cin), lambda n: (n, 0, 0, 0)),
            pl.BlockSpec((1, 1, RS, Cin), lambda n: (n, 1, 0, 0)),
            pl.BlockSpec((1, 1, RS, Cin), lambda n: (n, 2, 0, 0)),
            pl.BlockSpec((27 * Cin, Cout), lambda n: (0, 0)),
            pl.BlockSpec((1, Cout), lambda n: (0, 0)),
            pl.BlockSpec((1, Cout), lambda n: (0, 0)),
            pl.BlockSpec((1, 1, RS, Cout), lambda n: (n, 1, 0, 0)),
            pl.BlockSpec((Cout, NCp), lambda n: (0, 0)),
            pl.BlockSpec((1, NCp), lambda n: (0, 0)),
        ],
        out_specs=pl.BlockSpec((1, 1, NCp), lambda n: (n, 0, 0)),
        compiler_params=pltpu.CompilerParams(
            dimension_semantics=("parallel",),
            vmem_limit_bytes=100 * 1024 * 1024),
    )(x, x, x, wm, scale.reshape(1, Cout).astype(jnp.float32),
      shift.reshape(1, Cout).astype(jnp.float32), residual, fcw, fcb)
    return out[:, 0, :nc]


def _to_padded_flat(rows, N, T, H, W, C):
    Wp, Hp = W + 2, H + 2
    RS = _rs_of(H, W)
    x5 = rows.reshape(N, T, H, W, C)
    xp = jnp.pad(x5, ((0, 0), (1, 1), (1, 1), (1, 1), (0, 0)))
    flat = xp.reshape(N, T + 2, Hp * Wp, C)
    return jnp.pad(flat, ((0, 0), (0, 0), (0, RS - Hp * Wp), (0, 0)))


def _from_padded_flat(x_flat, N, T, H, W, C):
    Hp, Wp = H + 2, W + 2
    return x_flat[:, :, :Hp * Wp, :].reshape(N, T + 2, Hp, Wp, C)


def _im2col_strided(xp5, k, stride):
    N, Tp, Hp, Wp, C = xp5.shape
    oT = (Tp - k) // stride + 1
    oH = (Hp - k) // stride + 1
    oW = (Wp - k) // stride + 1
    patches = []
    for dt in range(k):
        for dh in range(k):
            for dw in range(k):
                patches.append(
                    xp5[:, dt:dt + (oT - 1) * stride + 1:stride,
                        dh:dh + (oH - 1) * stride + 1:stride,
                        dw:dw + (oW - 1) * stride + 1:stride, :])
    col = jnp.concatenate(patches, axis=-1)
    return col.reshape(N * oT * oH * oW, k * k * k * C), (N, oT, oH, oW)


def _strided_block0(x_flat, blk, dims_in, dims_out):
    N, T, H, W = dims_in
    _, oT, oH, oW = dims_out
    Cin = x_flat.shape[-1]
    Cout = blk['conv1_w'].shape[0]
    xp5 = _from_padded_flat(x_flat, N, T, H, W, Cin)
    col, _ = _im2col_strided(xp5, 3, 2)
    out1 = _matmul_bn(col, jnp.transpose(blk['conv1_w'], (2, 3, 4, 1, 0))
                      .reshape(27 * Cin, Cout),
                      blk['bn1_scale'], blk['bn1_shift'], relu=True)
    out1_flat = _to_padded_flat(out1, N, oT, oH, oW, Cout)
    sub = xp5[:, 1:1 + 2 * oT:2, 1:1 + 2 * oH:2, 1:1 + 2 * oW:2, :]
    sub = sub.reshape(N * oT * oH * oW, Cin)
    res = _matmul_bn(sub, blk['down_w'].reshape(Cout, Cin).T,
                     blk['down_bn_scale'], blk['down_bn_shift'], relu=False)
    res_flat = _to_padded_flat(res, N, oT, oH, oW, Cout)
    return _conv_s1(out1_flat, blk['conv2_w'], blk['bn2_scale'],
                    blk['bn2_shift'], H=oH, W=oW, residual=res_flat)


def kernel(x, stem_w, stem_scale, stem_shift,
           layer1_0_conv1_w, layer1_0_bn1_scale, layer1_0_bn1_shift,
           layer1_0_conv2_w, layer1_0_bn2_scale, layer1_0_bn2_shift,
           layer1_1_conv1_w, layer1_1_bn1_scale, layer1_1_bn1_shift,
           layer1_1_conv2_w, layer1_1_bn2_scale, layer1_1_bn2_shift,
           layer2_0_conv1_w, layer2_0_bn1_scale, layer2_0_bn1_shift,
           layer2_0_conv2_w, layer2_0_bn2_scale, layer2_0_bn2_shift,
           layer2_0_down_w, layer2_0_down_bn_scale, layer2_0_down_bn_shift,
           layer2_1_conv1_w, layer2_1_bn1_scale, layer2_1_bn1_shift,
           layer2_1_conv2_w, layer2_1_bn2_scale, layer2_1_bn2_shift,
           layer3_0_conv1_w, layer3_0_bn1_scale, layer3_0_bn1_shift,
           layer3_0_conv2_w, layer3_0_bn2_scale, layer3_0_bn2_shift,
           layer3_0_down_w, layer3_0_down_bn_scale, layer3_0_down_bn_shift,
           layer3_1_conv1_w, layer3_1_bn1_scale, layer3_1_bn1_shift,
           layer3_1_conv2_w, layer3_1_bn2_scale, layer3_1_bn2_shift,
           layer4_0_conv1_w, layer4_0_bn1_scale, layer4_0_bn1_shift,
           layer4_0_conv2_w, layer4_0_bn2_scale, layer4_0_bn2_shift,
           layer4_0_down_w, layer4_0_down_bn_scale, layer4_0_down_bn_shift,
           layer4_1_conv1_w, layer4_1_bn1_scale, layer4_1_bn1_shift,
           layer4_1_conv2_w, layer4_1_bn2_scale, layer4_1_bn2_shift,
           fc_w, fc_b):
    N = x.shape[0]
    xt = jnp.transpose(x, (0, 2, 3, 4, 1)).astype(jnp.bfloat16)
    xp = jnp.pad(xt, ((0, 0), (1, 1), (3, 3), (3, 3), (0, 0)))
    oT, oH, oW = 8, 56, 56
    patches = []
    for dt in range(3):
        for dh in range(7):
            for dw in range(7):
                patches.append(
                    xp[:, dt:dt + oT, dh:dh + 2 * oH - 1:2,
                       dw:dw + 2 * oW - 1:2, :])
    col = jnp.concatenate(patches, axis=-1).reshape(N * oT * oH * oW, 441)
    w_mat = jnp.transpose(stem_w, (2, 3, 4, 1, 0)).reshape(441, 64)
    stem_out = _matmul_bn(col, w_mat, stem_scale, stem_shift, relu=True)
    h = _to_padded_flat(stem_out, N, 8, 56, 56, 64)

    o1 = _conv_s1(h, layer1_0_conv1_w, layer1_0_bn1_scale, layer1_0_bn1_shift,
                  H=56, W=56)
    h = _conv_s1(o1, layer1_0_conv2_w, layer1_0_bn2_scale, layer1_0_bn2_shift,
                 H=56, W=56, residual=h)
    o1 = _conv_s1(h, layer1_1_conv1_w, layer1_1_bn1_scale, layer1_1_bn1_shift,
                  H=56, W=56)
    h = _conv_s1(o1, layer1_1_conv2_w, layer1_1_bn2_scale, layer1_1_bn2_shift,
                 H=56, W=56, residual=h)

    h = _strided_block0(
        h, {'conv1_w': layer2_0_conv1_w, 'bn1_scale': layer2_0_bn1_scale,
            'bn1_shift': layer2_0_bn1_shift, 'conv2_w': layer2_0_conv2_w,
            'bn2_scale': layer2_0_bn2_scale, 'bn2_shift': layer2_0_bn2_shift,
            'down_w': layer2_0_down_w, 'down_bn_scale': layer2_0_down_bn_scale,
            'down_bn_shift': layer2_0_down_bn_shift},
        (N, 8, 56, 56), (N, 4, 28, 28))
    o1 = _conv_s1(h, layer2_1_conv1_w, layer2_1_bn1_scale, layer2_1_bn1_shift,
                  H=28, W=28)
    h = _conv_s1(o1, layer2_1_conv2_w, layer2_1_bn2_scale, layer2_1_bn2_shift,
                 H=28, W=28, residual=h)

    return h[:, 0, :400, 0].astype(jnp.float32)
    h = _strided_block0(
        h, {'conv1_w': layer3_0_conv1_w, 'bn1_scale': layer3_0_bn1_scale,
            'bn1_shift': layer3_0_bn1_shift, 'conv2_w': layer3_0_conv2_w,
            'bn2_scale': layer3_0_bn2_scale, 'bn2_shift': layer3_0_bn2_shift,
            'down_w': layer3_0_down_w, 'down_bn_scale': layer3_0_down_bn_scale,
            'down_bn_shift': layer3_0_down_bn_shift},
        (N, 4, 28, 28), (N, 2, 14, 14))
    o1 = _conv_s1(h, layer3_1_conv1_w, layer3_1_bn1_scale, layer3_1_bn1_shift,
                  H=14, W=14)
    h = _conv_s1(o1, layer3_1_conv2_w, layer3_1_bn2_scale, layer3_1_bn2_shift,
                 H=14, W=14, residual=h)

    h = _strided_block0(
        h, {'conv1_w': layer4_0_conv1_w, 'bn1_scale': layer4_0_bn1_scale,
            'bn1_shift': layer4_0_bn1_shift, 'conv2_w': layer4_0_conv2_w,
            'bn2_scale': layer4_0_bn2_scale, 'bn2_shift': layer4_0_bn2_shift,
            'down_w': layer4_0_down_w, 'down_bn_scale': layer4_0_down_bn_scale,
            'down_bn_shift': layer4_0_down_bn_shift},
        (N, 2, 14, 14), (N, 1, 7, 7))
    o1 = _conv_s1(h, layer4_1_conv1_w, layer4_1_bn1_scale, layer4_1_bn1_shift,
                  H=7, W=7)
    logits = _conv_pool_fc(o1, layer4_1_conv2_w, layer4_1_bn2_scale,
                           layer4_1_bn2_shift, h, fc_w, fc_b, H=7, W=7)
    return logits


# speedup vs baseline: 1.5433x; 1.5433x over previous
import functools

import jax
import jax.numpy as jnp
from jax.experimental import pallas as pl
from jax.experimental.pallas import tpu as pltpu


def _rup(x, m):
    return (x + m - 1) // m * m


def _rs_of(H, W):
    base = (H + 2) * (W + 2)
    slack = 2 * (W + 2) + 2
    return _rup(base, 16) + slack


def _mmk(a_ref, b_ref, s_ref, t_ref, o_ref, *, relu):
    acc = jnp.dot(a_ref[...], b_ref[...], preferred_element_type=jnp.float32)
    y = acc * s_ref[...] + t_ref[...]
    if relu:
        y = jnp.maximum(y, 0.0)
    o_ref[...] = y.astype(o_ref.dtype)


def _matmul_bn(a, b, scale, shift, *, relu, tm=1024):
    M, K = a.shape
    N = b.shape[1]
    tm = min(tm, _rup(M, 8))
    Mp = _rup(M, tm)
    a = jnp.pad(a, ((0, Mp - M), (0, 0)))
    out = pl.pallas_call(
        functools.partial(_mmk, relu=relu),
        out_shape=jax.ShapeDtypeStruct((Mp, N), jnp.bfloat16),
        grid=(Mp // tm,),
        in_specs=[
            pl.BlockSpec((tm, K), lambda i: (i, 0)),
            pl.BlockSpec((K, N), lambda i: (0, 0)),
            pl.BlockSpec((1, N), lambda i: (0, 0)),
            pl.BlockSpec((1, N), lambda i: (0, 0)),
        ],
        out_specs=pl.BlockSpec((tm, N), lambda i: (i, 0)),
        compiler_params=pltpu.CompilerParams(
            dimension_semantics=("parallel",),
            vmem_limit_bytes=100 * 1024 * 1024),
    )(a, b.astype(jnp.bfloat16), scale.reshape(1, N).astype(jnp.float32),
      shift.reshape(1, N).astype(jnp.float32))
    return out[:M]


def _mmk_ta(a_ref, b_ref, s_ref, t_ref, o_ref, *, relu):
    acc = jax.lax.dot_general(a_ref[...], b_ref[...],
                              (((0,), (0,)), ((), ())),
                              preferred_element_type=jnp.float32)
    y = acc * s_ref[...] + t_ref[...]
    if relu:
        y = jnp.maximum(y, 0.0)
    o_ref[...] = y.astype(o_ref.dtype)


def _matmul_bn_ta(a_t, b, scale, shift, *, relu, tm=2048):
    K, M = a_t.shape
    N = b.shape[1]
    tm = min(tm, _rup(M, 128))
    Mp = _rup(M, tm)
    a_t = jnp.pad(a_t, ((0, 0), (0, Mp - M)))
    out = pl.pallas_call(
        functools.partial(_mmk_ta, relu=relu),
        out_shape=jax.ShapeDtypeStruct((Mp, N), jnp.bfloat16),
        grid=(Mp // tm,),
        in_specs=[
            pl.BlockSpec((K, tm), lambda i: (0, i)),
            pl.BlockSpec((K, N), lambda i: (0, 0)),
            pl.BlockSpec((1, N), lambda i: (0, 0)),
            pl.BlockSpec((1, N), lambda i: (0, 0)),
        ],
        out_specs=pl.BlockSpec((tm, N), lambda i: (i, 0)),
        compiler_params=pltpu.CompilerParams(
            dimension_semantics=("parallel",),
            vmem_limit_bytes=100 * 1024 * 1024),
    )(a_t, b.astype(jnp.bfloat16), scale.reshape(1, N).astype(jnp.float32),
      shift.reshape(1, N).astype(jnp.float32))
    return out[:M]


def _col_and_acc(x_refs, w_ref, *, M, Wp):
    pieces = []
    for x_ref in x_refs:
        xv = x_ref[0, 0]
        for dh in range(3):
            for dw in range(3):
                off = dh * Wp + dw
                pieces.append(xv[off:off + M, :])
    col = jnp.concatenate(pieces, axis=-1)
    return jnp.dot(col, w_ref[...], preferred_element_type=jnp.float32)


def _interior_mask(shape, *, H, W):
    Wp = W + 2
    r = jax.lax.broadcasted_iota(jnp.int32, shape, 0) + (Wp + 1)
    h = jnp.floor((r.astype(jnp.float32) + 0.5) *
                  jnp.float32(1.0 / Wp)).astype(jnp.int32)
    w = r - h * Wp
    return (h >= 1) & (h <= H) & (w >= 1) & (w <= W)


def _conv_s1_kernel(*refs, H, W, has_res, relu, Tp):
    if has_res:
        x0_ref, x1_ref, x2_ref, w_ref, s_ref, t_ref, res_ref, o_ref = refs
    else:
        x0_ref, x1_ref, x2_ref, w_ref, s_ref, t_ref, o_ref = refs
    tp = pl.program_id(1)
    Wp = W + 2
    RS = o_ref.shape[2]
    M = RS - (2 * Wp + 2)

    @pl.when(jnp.logical_or(tp == 0, tp == Tp - 1))
    def _():
        o_ref[...] = jnp.zeros_like(o_ref)

    @pl.when(jnp.logical_and(tp > 0, tp < Tp - 1))
    def _():
        D = Wp + 1
        acc = _col_and_acc((x0_ref, x1_ref, x2_ref), w_ref, M=M, Wp=Wp)
        y = acc * s_ref[...] + t_ref[...]
        if has_res:
            y = y + res_ref[0, 0, D:D + M, :].astype(jnp.float32)
        if relu:
            y = jnp.maximum(y, 0.0)
        y = jnp.where(_interior_mask(y.shape, H=H, W=W), y, 0.0)
        C = y.shape[1]
        o_ref[0, 0, 0:D, :] = jnp.zeros((D, C), o_ref.dtype)
        o_ref[0, 0, D:D + M, :] = y.astype(o_ref.dtype)
        o_ref[0, 0, D + M:RS, :] = jnp.zeros((RS - D - M, C), o_ref.dtype)


def _conv_s1(x, w, scale, shift, *, H, W, residual=None, relu=True):
    N, Tp, RS, Cin = x.shape
    Cout = w.shape[0]
    wm = jnp.transpose(w, (2, 3, 4, 1, 0)).reshape(27 * Cin, Cout)
    wm = wm.astype(jnp.bfloat16)
    sc = scale.reshape(1, Cout).astype(jnp.float32)
    sh = shift.reshape(1, Cout).astype(jnp.float32)
    in_specs = [
        pl.BlockSpec((1, 1, RS, Cin),
                     lambda n, t: (n, jnp.maximum(t - 1, 0), 0, 0)),
        pl.BlockSpec((1, 1, RS, Cin), lambda n, t: (n, t, 0, 0)),
        pl.BlockSpec((1, 1, RS, Cin),
                     lambda n, t: (n, jnp.minimum(t + 1, Tp - 1), 0, 0)),
        pl.BlockSpec((27 * Cin, Cout), lambda n, t: (0, 0)),
        pl.BlockSpec((1, Cout), lambda n, t: (0, 0)),
        pl.BlockSpec((1, Cout), lambda n, t: (0, 0)),
    ]
    args = [x, x, x, wm, sc, sh]
    if residual is not None:
        in_specs.append(pl.BlockSpec((1, 1, RS, Cout),
                                     lambda n, t: (n, t, 0, 0)))
        args.append(residual)
    kern = functools.partial(_conv_s1_kernel, H=H, W=W,
                             has_res=residual is not None, relu=relu, Tp=Tp)
    return pl.pallas_call(
        kern,
        out_shape=jax.ShapeDtypeStruct((N, Tp, RS, Cout), jnp.bfloat16),
        grid=(N, Tp),
        in_specs=in_specs,
        out_specs=pl.BlockSpec((1, 1, RS, Cout), lambda n, t: (n, t, 0, 0)),
        compiler_params=pltpu.CompilerParams(
            dimension_semantics=("parallel", "arbitrary"),
            vmem_limit_bytes=100 * 1024 * 1024),
    )(*args)


def _phase_split(x_flat, N, T, H, W, C, oH, oW):
    Hp, Wp = H + 2, W + 2
    Tpi = T + 2
    Wq = oW + 2
    M = _rup((oH + 2) * (oW + 2), 16)
    qmax = M + Wq + 2
    Hq = max((qmax + Wq - 1) // Wq + 1, (Hp + 1) // 2)
    RQ = _rup(Hq * Wq, 16)
    x5 = x_flat[:, :, :Hp * Wp, :].reshape(N, Tpi, Hp, Wp, C)
    x5 = jnp.pad(x5, ((0, 0), (0, 0), (0, 2 * Hq - Hp), (0, 2 * Wq - Wp),
                      (0, 0)))
    x5 = x5.reshape(N, Tpi, Hq, 2, Wq, 2, C)
    x5 = jnp.transpose(x5, (0, 1, 3, 5, 2, 4, 6))
    xq = x5.reshape(N, Tpi, 4, Hq * Wq, C)
    return jnp.pad(xq, ((0, 0), (0, 0), (0, 0), (0, RQ - Hq * Wq), (0, 0)))


def _conv_s2_kernel(x0_ref, x1_ref, x2_ref, w_ref, s_ref, t_ref,
                    wd_ref, ds_ref, dt_ref, o_ref, r_ref, *, oH, oW, Tpo):
    ts = pl.program_id(1)
    oWp = oW + 2
    RS = o_ref.shape[2]
    M = RS - (2 * oWp + 2)
    D = oWp + 1

    @pl.when(jnp.logical_or(ts == 0, ts == Tpo - 1))
    def _():
        o_ref[...] = jnp.zeros_like(o_ref)
        r_ref[...] = jnp.zeros_like(r_ref)

    @pl.when(jnp.logical_and(ts > 0, ts < Tpo - 1))
    def _():
        pieces = []
        for x_ref in (x0_ref, x1_ref, x2_ref):
            xv = x_ref[0, 0]
            for dh in range(3):
                for dw in range(3):
                    ph = (dh % 2) * 2 + (dw % 2)
                    off = (dh // 2) * oWp + (dw // 2)
                    pieces.append(xv[ph, off:off + M, :])
        col = jnp.concatenate(pieces, axis=-1)
        acc = jnp.dot(col, w_ref[...], preferred_element_type=jnp.float32)
        y = acc * s_ref[...] + t_ref[...]
        y = jnp.maximum(y, 0.0)
        mask = _interior_mask(y.shape, H=oH, W=oW)
        y = jnp.where(mask, y, 0.0)
        C = y.shape[1]
        o_ref[0, 0, 0:D, :] = jnp.zeros((D, C), o_ref.dtype)
        o_ref[0, 0, D:D + M, :] = y.astype(o_ref.dtype)
        o_ref[0, 0, D + M:RS, :] = jnp.zeros((RS - D - M, C), o_ref.dtype)
        accd = jnp.dot(x1_ref[0, 0, 3, 0:M, :], wd_ref[...],
                       preferred_element_type=jnp.float32)
        yr = accd * ds_ref[...] + dt_ref[...]
        yr = jnp.where(mask, yr, 0.0)
        r_ref[0, 0, 0:D, :] = jnp.zeros((D, C), r_ref.dtype)
        r_ref[0, 0, D:D + M, :] = yr.astype(r_ref.dtype)
        r_ref[0, 0, D + M:RS, :] = jnp.zeros((RS - D - M, C), r_ref.dtype)


def _conv_s2(x_flat, blk, dims_in, dims_out):
    N, T, H, W = dims_in
    _, oT, oH, oW = dims_out
    Cin = x_flat.shape[-1]
    Cout = blk['conv1_w'].shape[0]
    Tpi, Tpo = T + 2, oT + 2
    xq = _phase_split(x_flat, N, T, H, W, Cin, oH, oW)
    RQ = xq.shape[3]
    RSo = _rs_of(oH, oW)
    wm = jnp.transpose(blk['conv1_w'], (2, 3, 4, 1, 0)
                       ).reshape(27 * Cin, Cout).astype(jnp.bfloat16)
    wd = blk['down_w'].reshape(Cout, Cin).T.astype(jnp.bfloat16)
    sspec = pl.BlockSpec((1, Cout), lambda n, t: (0, 0))
    out1, res = pl.pallas_call(
        functools.partial(_conv_s2_kernel, oH=oH, oW=oW, Tpo=Tpo),
        out_shape=(jax.ShapeDtypeStruct((N, Tpo, RSo, Cout), jnp.bfloat16),
                   jax.ShapeDtypeStruct((N, Tpo, RSo, Cout), jnp.bfloat16)),
        grid=(N, Tpo),
        in_specs=[
            pl.BlockSpec((1, 1, 4, RQ, Cin),
                         lambda n, t: (n, jnp.clip(2 * t - 2, 0, Tpi - 1),
                                       0, 0, 0)),
            pl.BlockSpec((1, 1, 4, RQ, Cin),
                         lambda n, t: (n, jnp.clip(2 * t - 1, 0, Tpi - 1),
                                       0, 0, 0)),
            pl.BlockSpec((1, 1, 4, RQ, Cin),
                         lambda n, t: (n, jnp.clip(2 * t, 0, Tpi - 1),
                                       0, 0, 0)),
            pl.BlockSpec((27 * Cin, Cout), lambda n, t: (0, 0)),
            sspec, sspec,
            pl.BlockSpec((Cin, Cout), lambda n, t: (0, 0)),
            sspec, sspec,
        ],
        out_specs=(pl.BlockSpec((1, 1, RSo, Cout), lambda n, t: (n, t, 0, 0)),
                   pl.BlockSpec((1, 1, RSo, Cout),
                                lambda n, t: (n, t, 0, 0))),
        compiler_params=pltpu.CompilerParams(
            dimension_semantics=("parallel", "arbitrary"),
            vmem_limit_bytes=100 * 1024 * 1024),
    )(xq, xq, xq, wm,
      blk['bn1_scale'].reshape(1, Cout).astype(jnp.float32),
      blk['bn1_shift'].reshape(1, Cout).astype(jnp.float32),
      wd,
      blk['down_bn_scale'].reshape(1, Cout).astype(jnp.float32),
      blk['down_bn_shift'].reshape(1, Cout).astype(jnp.float32))
    return out1, res


def _conv_pool_fc_kernel(x0_ref, x1_ref, x2_ref, w_ref, s_ref, t_ref,
                         res_ref, fcw_ref, fcb_ref, o_ref, *, H, W):
    Wp = W + 2
    RS = x1_ref.shape[2]
    M = RS - (2 * Wp + 2)
    D = Wp + 1
    acc = _col_and_acc((x0_ref, x1_ref, x2_ref), w_ref, M=M, Wp=Wp)
    y = acc * s_ref[...] + t_ref[...]
    y = y + res_ref[0, 0, D:D + M, :].astype(jnp.float32)
    y = jnp.maximum(y, 0.0)
    y = jnp.where(_interior_mask(y.shape, H=H, W=W), y, 0.0)
    pooled = jnp.sum(y, axis=0, keepdims=True) * (1.0 / (H * W))
    o_ref[0] = (jnp.dot(pooled, fcw_ref[...],
                        preferred_element_type=jnp.float32) + fcb_ref[...])


def _conv_pool_fc(x, w, scale, shift, residual, fc_w, fc_b, *, H, W):
    N, Tp, RS, Cin = x.shape
    Cout = w.shape[0]
    nc = fc_w.shape[1]
    NCp = _rup(nc, 128)
    wm = jnp.transpose(w, (2, 3, 4, 1, 0)).reshape(27 * Cin, Cout)
    wm = wm.astype(jnp.bfloat16)
    fcw = jnp.pad(fc_w.astype(jnp.float32), ((0, 0), (0, NCp - nc)))
    fcb = jnp.pad(fc_b.astype(jnp.float32), (0, NCp - nc)).reshape(1, NCp)
    out = pl.pallas_call(
        functools.partial(_conv_pool_fc_kernel, H=H, W=W),
        out_shape=jax.ShapeDtypeStruct((N, 1, NCp), jnp.float32),
        grid=(N,),
        in_specs=[
            pl.BlockSpec((1, 1, RS, Cin), lambda n: (n, 0, 0, 0)),
            pl.BlockSpec((1, 1, RS, Cin), lambda n: (n, 1, 0, 0)),
            pl.BlockSpec((1, 1, RS, Cin), lambda n: (n, 2, 0, 0)),
            pl.BlockSpec((27 * Cin, Cout), lambda n: (0, 0)),
            pl.BlockSpec((1, Cout), lambda n: (0, 0)),
            pl.BlockSpec((1, Cout), lambda n: (0, 0)),
            pl.BlockSpec((1, 1, RS, Cout), lambda n: (n, 1, 0, 0)),
            pl.BlockSpec((Cout, NCp), lambda n: (0, 0)),
            pl.BlockSpec((1, NCp), lambda n: (0, 0)),
        ],
        out_specs=pl.BlockSpec((1, 1, NCp), lambda n: (n, 0, 0)),
        compiler_params=pltpu.CompilerParams(
            dimension_semantics=("parallel",),
            vmem_limit_bytes=100 * 1024 * 1024),
    )(x, x, x, wm, scale.reshape(1, Cout).astype(jnp.float32),
      shift.reshape(1, Cout).astype(jnp.float32), residual, fcw, fcb)
    return out[:, 0, :nc]


def _to_padded_flat(rows, N, T, H, W, C):
    Wp, Hp = W + 2, H + 2
    RS = _rs_of(H, W)
    x5 = rows.reshape(N, T, H, W, C)
    xp = jnp.pad(x5, ((0, 0), (1, 1), (1, 1), (1, 1), (0, 0)))
    flat = xp.reshape(N, T + 2, Hp * Wp, C)
    return jnp.pad(flat, ((0, 0), (0, 0), (0, RS - Hp * Wp), (0, 0)))


def _from_padded_flat(x_flat, N, T, H, W, C):
    Hp, Wp = H + 2, W + 2
    return x_flat[:, :, :Hp * Wp, :].reshape(N, T + 2, Hp, Wp, C)


def _im2col_strided(xp5, k, stride):
    N, Tp, Hp, Wp, C = xp5.shape
    oT = (Tp - k) // stride + 1
    oH = (Hp - k) // stride + 1
    oW = (Wp - k) // stride + 1
    patches = []
    for dt in range(k):
        for dh in range(k):
            for dw in range(k):
                patches.append(
                    xp5[:, dt:dt + (oT - 1) * stride + 1:stride,
                        dh:dh + (oH - 1) * stride + 1:stride,
                        dw:dw + (oW - 1) * stride + 1:stride, :])
    col = jnp.concatenate(patches, axis=-1)
    return col.reshape(N * oT * oH * oW, k * k * k * C), (N, oT, oH, oW)


def _strided_block0(x_flat, blk, dims_in, dims_out):
    _, oT, oH, oW = dims_out
    out1_flat, res_flat = _conv_s2(x_flat, blk, dims_in, dims_out)
    return _conv_s1(out1_flat, blk['conv2_w'], blk['bn2_scale'],
                    blk['bn2_shift'], H=oH, W=oW, residual=res_flat)


def kernel(x, stem_w, stem_scale, stem_shift,
           layer1_0_conv1_w, layer1_0_bn1_scale, layer1_0_bn1_shift,
           layer1_0_conv2_w, layer1_0_bn2_scale, layer1_0_bn2_shift,
           layer1_1_conv1_w, layer1_1_bn1_scale, layer1_1_bn1_shift,
           layer1_1_conv2_w, layer1_1_bn2_scale, layer1_1_bn2_shift,
           layer2_0_conv1_w, layer2_0_bn1_scale, layer2_0_bn1_shift,
           layer2_0_conv2_w, layer2_0_bn2_scale, layer2_0_bn2_shift,
           layer2_0_down_w, layer2_0_down_bn_scale, layer2_0_down_bn_shift,
           layer2_1_conv1_w, layer2_1_bn1_scale, layer2_1_bn1_shift,
           layer2_1_conv2_w, layer2_1_bn2_scale, layer2_1_bn2_shift,
           layer3_0_conv1_w, layer3_0_bn1_scale, layer3_0_bn1_shift,
           layer3_0_conv2_w, layer3_0_bn2_scale, layer3_0_bn2_shift,
           layer3_0_down_w, layer3_0_down_bn_scale, layer3_0_down_bn_shift,
           layer3_1_conv1_w, layer3_1_bn1_scale, layer3_1_bn1_shift,
           layer3_1_conv2_w, layer3_1_bn2_scale, layer3_1_bn2_shift,
           layer4_0_conv1_w, layer4_0_bn1_scale, layer4_0_bn1_shift,
           layer4_0_conv2_w, layer4_0_bn2_scale, layer4_0_bn2_shift,
           layer4_0_down_w, layer4_0_down_bn_scale, layer4_0_down_bn_shift,
           layer4_1_conv1_w, layer4_1_bn1_scale, layer4_1_bn1_shift,
           layer4_1_conv2_w, layer4_1_bn2_scale, layer4_1_bn2_shift,
           fc_w, fc_b):
    N = x.shape[0]
    oT, oH, oW = 8, 56, 56
    xp = jnp.pad(x.astype(jnp.bfloat16),
                 ((0, 0), (0, 0), (1, 1), (3, 3), (3, 3)))
    xq = xp.reshape(N, 3, 10, 59, 2, 59, 2)
    xq = jnp.transpose(xq, (0, 1, 4, 6, 2, 3, 5))
    patches = []
    for dt in range(3):
        for dh in range(7):
            for dw in range(7):
                for c in range(3):
                    patches.append(
                        xq[:, c, dh % 2, dw % 2, dt:dt + oT,
                           dh // 2:dh // 2 + oH, dw // 2:dw // 2 + oW])
    col_t = jnp.stack(patches).reshape(441, N * oT * oH * oW)
    w_mat = jnp.transpose(stem_w, (2, 3, 4, 1, 0)).reshape(441, 64)
    stem_out = _matmul_bn_ta(col_t, w_mat, stem_scale, stem_shift, relu=True)
    h = _to_padded_flat(stem_out, N, 8, 56, 56, 64)

    o1 = _conv_s1(h, layer1_0_conv1_w, layer1_0_bn1_scale, layer1_0_bn1_shift,
                  H=56, W=56)
    h = _conv_s1(o1, layer1_0_conv2_w, layer1_0_bn2_scale, layer1_0_bn2_shift,
                 H=56, W=56, residual=h)
    o1 = _conv_s1(h, layer1_1_conv1_w, layer1_1_bn1_scale, layer1_1_bn1_shift,
                  H=56, W=56)
    h = _conv_s1(o1, layer1_1_conv2_w, layer1_1_bn2_scale, layer1_1_bn2_shift,
                 H=56, W=56, residual=h)

    h = _strided_block0(
        h, {'conv1_w': layer2_0_conv1_w, 'bn1_scale': layer2_0_bn1_scale,
            'bn1_shift': layer2_0_bn1_shift, 'conv2_w': layer2_0_conv2_w,
            'bn2_scale': layer2_0_bn2_scale, 'bn2_shift': layer2_0_bn2_shift,
            'down_w': layer2_0_down_w, 'down_bn_scale': layer2_0_down_bn_scale,
            'down_bn_shift': layer2_0_down_bn_shift},
        (N, 8, 56, 56), (N, 4, 28, 28))
    o1 = _conv_s1(h, layer2_1_conv1_w, layer2_1_bn1_scale, layer2_1_bn1_shift,
                  H=28, W=28)
    h = _conv_s1(o1, layer2_1_conv2_w, layer2_1_bn2_scale, layer2_1_bn2_shift,
                 H=28, W=28, residual=h)

    h = _strided_block0(
        h, {'conv1_w': layer3_0_conv1_w, 'bn1_scale': layer3_0_bn1_scale,
            'bn1_shift': layer3_0_bn1_shift, 'conv2_w': layer3_0_conv2_w,
            'bn2_scale': layer3_0_bn2_scale, 'bn2_shift': layer3_0_bn2_shift,
            'down_w': layer3_0_down_w, 'down_bn_scale': layer3_0_down_bn_scale,
            'down_bn_shift': layer3_0_down_bn_shift},
        (N, 4, 28, 28), (N, 2, 14, 14))
    o1 = _conv_s1(h, layer3_1_conv1_w, layer3_1_bn1_scale, layer3_1_bn1_shift,
                  H=14, W=14)
    h = _conv_s1(o1, layer3_1_conv2_w, layer3_1_bn2_scale, layer3_1_bn2_shift,
                 H=14, W=14, residual=h)

    h = _strided_block0(
        h, {'conv1_w': layer4_0_conv1_w, 'bn1_scale': layer4_0_bn1_scale,
            'bn1_shift': layer4_0_bn1_shift, 'conv2_w': layer4_0_conv2_w,
            'bn2_scale': layer4_0_bn2_scale, 'bn2_shift': layer4_0_bn2_shift,
            'down_w': layer4_0_down_w, 'down_bn_scale': layer4_0_down_bn_scale,
            'down_bn_shift': layer4_0_down_bn_shift},
        (N, 2, 14, 14), (N, 1, 7, 7))
    o1 = _conv_s1(h, layer4_1_conv1_w, layer4_1_bn1_scale, layer4_1_bn1_shift,
                  H=7, W=7)
    logits = _conv_pool_fc(o1, layer4_1_conv2_w, layer4_1_bn2_scale,
                           layer4_1_bn2_shift, h, fc_w, fc_b, H=7, W=7)
    return logits


# speedup vs baseline: 1.8918x; 1.2258x over previous
import functools

import jax
import jax.numpy as jnp
from jax.experimental import pallas as pl
from jax.experimental.pallas import tpu as pltpu


def _rup(x, m):
    return (x + m - 1) // m * m


def _rs_of(H, W):
    base = (H + 2) * (W + 2)
    slack = 2 * (W + 2) + 2
    return _rup(base, 16) + slack


def _mmk(a_ref, b_ref, s_ref, t_ref, o_ref, *, relu):
    acc = jnp.dot(a_ref[...], b_ref[...], preferred_element_type=jnp.float32)
    y = acc * s_ref[...] + t_ref[...]
    if relu:
        y = jnp.maximum(y, 0.0)
    o_ref[...] = y.astype(o_ref.dtype)


def _matmul_bn(a, b, scale, shift, *, relu, tm=1024):
    M, K = a.shape
    N = b.shape[1]
    tm = min(tm, _rup(M, 8))
    Mp = _rup(M, tm)
    a = jnp.pad(a, ((0, Mp - M), (0, 0)))
    out = pl.pallas_call(
        functools.partial(_mmk, relu=relu),
        out_shape=jax.ShapeDtypeStruct((Mp, N), jnp.bfloat16),
        grid=(Mp // tm,),
        in_specs=[
            pl.BlockSpec((tm, K), lambda i: (i, 0)),
            pl.BlockSpec((K, N), lambda i: (0, 0)),
            pl.BlockSpec((1, N), lambda i: (0, 0)),
            pl.BlockSpec((1, N), lambda i: (0, 0)),
        ],
        out_specs=pl.BlockSpec((tm, N), lambda i: (i, 0)),
        compiler_params=pltpu.CompilerParams(
            dimension_semantics=("parallel",),
            vmem_limit_bytes=100 * 1024 * 1024),
    )(a, b.astype(jnp.bfloat16), scale.reshape(1, N).astype(jnp.float32),
      shift.reshape(1, N).astype(jnp.float32))
    return out[:M]


def _mmk_ta(a_ref, b_ref, s_ref, t_ref, o_ref, *, relu):
    acc = jax.lax.dot_general(a_ref[...], b_ref[...],
                              (((0,), (0,)), ((), ())),
                              preferred_element_type=jnp.float32)
    y = acc * s_ref[...] + t_ref[...]
    if relu:
        y = jnp.maximum(y, 0.0)
    o_ref[...] = y.astype(o_ref.dtype)


def _matmul_bn_ta(a_t, b, scale, shift, *, relu, tm=2048):
    K, M = a_t.shape
    N = b.shape[1]
    tm = min(tm, _rup(M, 128))
    Mp = _rup(M, tm)
    a_t = jnp.pad(a_t, ((0, 0), (0, Mp - M)))
    out = pl.pallas_call(
        functools.partial(_mmk_ta, relu=relu),
        out_shape=jax.ShapeDtypeStruct((Mp, N), jnp.bfloat16),
        grid=(Mp // tm,),
        in_specs=[
            pl.BlockSpec((K, tm), lambda i: (0, i)),
            pl.BlockSpec((K, N), lambda i: (0, 0)),
            pl.BlockSpec((1, N), lambda i: (0, 0)),
            pl.BlockSpec((1, N), lambda i: (0, 0)),
        ],
        out_specs=pl.BlockSpec((tm, N), lambda i: (i, 0)),
        compiler_params=pltpu.CompilerParams(
            dimension_semantics=("parallel",),
            vmem_limit_bytes=100 * 1024 * 1024),
    )(a_t, b.astype(jnp.bfloat16), scale.reshape(1, N).astype(jnp.float32),
      shift.reshape(1, N).astype(jnp.float32))
    return out[:M]


def _col_and_acc(x_refs, w_ref, *, M, Wp):
    pieces = []
    for x_ref in x_refs:
        xv = x_ref[0, 0]
        for dh in range(3):
            for dw in range(3):
                off = dh * Wp + dw
                pieces.append(xv[off:off + M, :])
    col = jnp.concatenate(pieces, axis=-1)
    return jnp.dot(col, w_ref[...], preferred_element_type=jnp.float32)


def _interior_mask(shape, *, H, W):
    Wp = W + 2
    r = jax.lax.broadcasted_iota(jnp.int32, shape, 0) + (Wp + 1)
    h = jnp.floor((r.astype(jnp.float32) + 0.5) *
                  jnp.float32(1.0 / Wp)).astype(jnp.int32)
    w = r - h * Wp
    return (h >= 1) & (h <= H) & (w >= 1) & (w <= W)


def _conv_s1_kernel(*refs, H, W, has_res, relu, Tp):
    if has_res:
        x0_ref, x1_ref, x2_ref, w_ref, s_ref, t_ref, res_ref, o_ref = refs
    else:
        x0_ref, x1_ref, x2_ref, w_ref, s_ref, t_ref, o_ref = refs
    tp = pl.program_id(1)
    Wp = W + 2
    RS = o_ref.shape[2]
    M = RS - (2 * Wp + 2)

    @pl.when(jnp.logical_or(tp == 0, tp == Tp - 1))
    def _():
        o_ref[...] = jnp.zeros_like(o_ref)

    @pl.when(jnp.logical_and(tp > 0, tp < Tp - 1))
    def _():
        D = Wp + 1
        acc = _col_and_acc((x0_ref, x1_ref, x2_ref), w_ref, M=M, Wp=Wp)
        y = acc * s_ref[...] + t_ref[...]
        if has_res:
            y = y + res_ref[0, 0, D:D + M, :].astype(jnp.float32)
        if relu:
            y = jnp.maximum(y, 0.0)
        y = jnp.where(_interior_mask(y.shape, H=H, W=W), y, 0.0)
        C = y.shape[1]
        o_ref[0, 0, 0:D, :] = jnp.zeros((D, C), o_ref.dtype)
        o_ref[0, 0, D:D + M, :] = y.astype(o_ref.dtype)
        o_ref[0, 0, D + M:RS, :] = jnp.zeros((RS - D - M, C), o_ref.dtype)


def _conv_s1(x, w, scale, shift, *, H, W, residual=None, relu=True):
    N, Tp, RS, Cin = x.shape
    Cout = w.shape[0]
    wm = jnp.transpose(w, (2, 3, 4, 1, 0)).reshape(27 * Cin, Cout)
    wm = wm.astype(jnp.bfloat16)
    sc = scale.reshape(1, Cout).astype(jnp.float32)
    sh = shift.reshape(1, Cout).astype(jnp.float32)
    in_specs = [
        pl.BlockSpec((1, 1, RS, Cin),
                     lambda n, t: (n, jnp.maximum(t - 1, 0), 0, 0)),
        pl.BlockSpec((1, 1, RS, Cin), lambda n, t: (n, t, 0, 0)),
        pl.BlockSpec((1, 1, RS, Cin),
                     lambda n, t: (n, jnp.minimum(t + 1, Tp - 1), 0, 0)),
        pl.BlockSpec((27 * Cin, Cout), lambda n, t: (0, 0)),
        pl.BlockSpec((1, Cout), lambda n, t: (0, 0)),
        pl.BlockSpec((1, Cout), lambda n, t: (0, 0)),
    ]
    args = [x, x, x, wm, sc, sh]
    if residual is not None:
        in_specs.append(pl.BlockSpec((1, 1, RS, Cout),
                                     lambda n, t: (n, t, 0, 0)))
        args.append(residual)
    kern = functools.partial(_conv_s1_kernel, H=H, W=W,
                             has_res=residual is not None, relu=relu, Tp=Tp)
    return pl.pallas_call(
        kern,
        out_shape=jax.ShapeDtypeStruct((N, Tp, RS, Cout), jnp.bfloat16),
        grid=(N, Tp),
        in_specs=in_specs,
        out_specs=pl.BlockSpec((1, 1, RS, Cout), lambda n, t: (n, t, 0, 0)),
        compiler_params=pltpu.CompilerParams(
            dimension_semantics=("parallel", "arbitrary"),
            vmem_limit_bytes=100 * 1024 * 1024),
    )(*args)


def _phase_split(x_flat, N, T, H, W, C, oH, oW):
    Hp, Wp = H + 2, W + 2
    Tpi = T + 2
    Wq = oW + 2
    M = _rup((oH + 2) * (oW + 2), 16)
    qmax = M + Wq + 2
    Hq = max((qmax + Wq - 1) // Wq + 1, (Hp + 1) // 2)
    RQ = _rup(Hq * Wq, 16)
    x5 = x_flat[:, :, :Hp * Wp, :].reshape(N, Tpi, Hp, Wp, C)
    x5 = jnp.pad(x5, ((0, 0), (0, 0), (0, 2 * Hq - Hp), (0, 2 * Wq - Wp),
                      (0, 0)))
    x5 = x5.reshape(N, Tpi, Hq, 2, Wq, 2, C)
    x5 = jnp.transpose(x5, (0, 1, 3, 5, 2, 4, 6))
    xq = x5.reshape(N, Tpi, 4, Hq * Wq, C)
    return jnp.pad(xq, ((0, 0), (0, 0), (0, 0), (0, RQ - Hq * Wq), (0, 0)))


def _conv_s2_kernel(x0_ref, x1_ref, x2_ref, w_ref, s_ref, t_ref,
                    wd_ref, ds_ref, dt_ref, o_ref, r_ref, *, oH, oW, Tpo):
    ts = pl.program_id(1)
    oWp = oW + 2
    RS = o_ref.shape[2]
    M = RS - (2 * oWp + 2)
    D = oWp + 1

    @pl.when(jnp.logical_or(ts == 0, ts == Tpo - 1))
    def _():
        o_ref[...] = jnp.zeros_like(o_ref)
        r_ref[...] = jnp.zeros_like(r_ref)

    @pl.when(jnp.logical_and(ts > 0, ts < Tpo - 1))
    def _():
        pieces = []
        for x_ref in (x0_ref, x1_ref, x2_ref):
            xv = x_ref[0, 0]
            for dh in range(3):
                for dw in range(3):
                    ph = (dh % 2) * 2 + (dw % 2)
                    off = (dh // 2) * oWp + (dw // 2)
                    pieces.append(xv[ph, off:off + M, :])
        col = jnp.concatenate(pieces, axis=-1)
        acc = jnp.dot(col, w_ref[...], preferred_element_type=jnp.float32)
        y = acc * s_ref[...] + t_ref[...]
        y = jnp.maximum(y, 0.0)
        mask = _interior_mask(y.shape, H=oH, W=oW)
        y = jnp.where(mask, y, 0.0)
        C = y.shape[1]
        o_ref[0, 0, 0:D, :] = jnp.zeros((D, C), o_ref.dtype)
        o_ref[0, 0, D:D + M, :] = y.astype(o_ref.dtype)
        o_ref[0, 0, D + M:RS, :] = jnp.zeros((RS - D - M, C), o_ref.dtype)
        accd = jnp.dot(x1_ref[0, 0, 3, 0:M, :], wd_ref[...],
                       preferred_element_type=jnp.float32)
        yr = accd * ds_ref[...] + dt_ref[...]
        yr = jnp.where(mask, yr, 0.0)
        r_ref[0, 0, 0:D, :] = jnp.zeros((D, C), r_ref.dtype)
        r_ref[0, 0, D:D + M, :] = yr.astype(r_ref.dtype)
        r_ref[0, 0, D + M:RS, :] = jnp.zeros((RS - D - M, C), r_ref.dtype)


def _conv_s2(x_flat, blk, dims_in, dims_out):
    N, T, H, W = dims_in
    _, oT, oH, oW = dims_out
    Cin = x_flat.shape[-1]
    Cout = blk['conv1_w'].shape[0]
    Tpi, Tpo = T + 2, oT + 2
    xq = _phase_split(x_flat, N, T, H, W, Cin, oH, oW)
    RQ = xq.shape[3]
    RSo = _rs_of(oH, oW)
    wm = jnp.transpose(blk['conv1_w'], (2, 3, 4, 1, 0)
                       ).reshape(27 * Cin, Cout).astype(jnp.bfloat16)
    wd = blk['down_w'].reshape(Cout, Cin).T.astype(jnp.bfloat16)
    sspec = pl.BlockSpec((1, Cout), lambda n, t: (0, 0))
    out1, res = pl.pallas_call(
        functools.partial(_conv_s2_kernel, oH=oH, oW=oW, Tpo=Tpo),
        out_shape=(jax.ShapeDtypeStruct((N, Tpo, RSo, Cout), jnp.bfloat16),
                   jax.ShapeDtypeStruct((N, Tpo, RSo, Cout), jnp.bfloat16)),
        grid=(N, Tpo),
        in_specs=[
            pl.BlockSpec((1, 1, 4, RQ, Cin),
                         lambda n, t: (n, jnp.clip(2 * t - 2, 0, Tpi - 1),
                                       0, 0, 0)),
            pl.BlockSpec((1, 1, 4, RQ, Cin),
                         lambda n, t: (n, jnp.clip(2 * t - 1, 0, Tpi - 1),
                                       0, 0, 0)),
            pl.BlockSpec((1, 1, 4, RQ, Cin),
                         lambda n, t: (n, jnp.clip(2 * t, 0, Tpi - 1),
                                       0, 0, 0)),
            pl.BlockSpec((27 * Cin, Cout), lambda n, t: (0, 0)),
            sspec, sspec,
            pl.BlockSpec((Cin, Cout), lambda n, t: (0, 0)),
            sspec, sspec,
        ],
        out_specs=(pl.BlockSpec((1, 1, RSo, Cout), lambda n, t: (n, t, 0, 0)),
                   pl.BlockSpec((1, 1, RSo, Cout),
                                lambda n, t: (n, t, 0, 0))),
        compiler_params=pltpu.CompilerParams(
            dimension_semantics=("parallel", "arbitrary"),
            vmem_limit_bytes=100 * 1024 * 1024),
    )(xq, xq, xq, wm,
      blk['bn1_scale'].reshape(1, Cout).astype(jnp.float32),
      blk['bn1_shift'].reshape(1, Cout).astype(jnp.float32),
      wd,
      blk['down_bn_scale'].reshape(1, Cout).astype(jnp.float32),
      blk['down_bn_shift'].reshape(1, Cout).astype(jnp.float32))
    return out1, res


def _conv_pool_fc_kernel(x0_ref, x1_ref, x2_ref, w_ref, s_ref, t_ref,
                         res_ref, fcw_ref, fcb_ref, o_ref, *, H, W):
    Wp = W + 2
    RS = x1_ref.shape[2]
    M = RS - (2 * Wp + 2)
    D = Wp + 1
    acc = _col_and_acc((x0_ref, x1_ref, x2_ref), w_ref, M=M, Wp=Wp)
    y = acc * s_ref[...] + t_ref[...]
    y = y + res_ref[0, 0, D:D + M, :].astype(jnp.float32)
    y = jnp.maximum(y, 0.0)
    y = jnp.where(_interior_mask(y.shape, H=H, W=W), y, 0.0)
    pooled = jnp.sum(y, axis=0, keepdims=True) * (1.0 / (H * W))
    o_ref[0] = (jnp.dot(pooled, fcw_ref[...],
                        preferred_element_type=jnp.float32) + fcb_ref[...])


def _conv_pool_fc(x, w, scale, shift, residual, fc_w, fc_b, *, H, W):
    N, Tp, RS, Cin = x.shape
    Cout = w.shape[0]
    nc = fc_w.shape[1]
    NCp = _rup(nc, 128)
    wm = jnp.transpose(w, (2, 3, 4, 1, 0)).reshape(27 * Cin, Cout)
    wm = wm.astype(jnp.bfloat16)
    fcw = jnp.pad(fc_w.astype(jnp.float32), ((0, 0), (0, NCp - nc)))
    fcb = jnp.pad(fc_b.astype(jnp.float32), (0, NCp - nc)).reshape(1, NCp)
    out = pl.pallas_call(
        functools.partial(_conv_pool_fc_kernel, H=H, W=W),
        out_shape=jax.ShapeDtypeStruct((N, 1, NCp), jnp.float32),
        grid=(N,),
        in_specs=[
            pl.BlockSpec((1, 1, RS, Cin), lambda n: (n, 0, 0, 0)),
            pl.BlockSpec((1, 1, RS, Cin), lambda n: (n, 1, 0, 0)),
            pl.BlockSpec((1, 1, RS, Cin), lambda n: (n, 2, 0, 0)),
            pl.BlockSpec((27 * Cin, Cout), lambda n: (0, 0)),
            pl.BlockSpec((1, Cout), lambda n: (0, 0)),
            pl.BlockSpec((1, Cout), lambda n: (0, 0)),
            pl.BlockSpec((1, 1, RS, Cout), lambda n: (n, 1, 0, 0)),
            pl.BlockSpec((Cout, NCp), lambda n: (0, 0)),
            pl.BlockSpec((1, NCp), lambda n: (0, 0)),
        ],
        out_specs=pl.BlockSpec((1, 1, NCp), lambda n: (n, 0, 0)),
        compiler_params=pltpu.CompilerParams(
            dimension_semantics=("parallel",),
            vmem_limit_bytes=100 * 1024 * 1024),
    )(x, x, x, wm, scale.reshape(1, Cout).astype(jnp.float32),
      shift.reshape(1, Cout).astype(jnp.float32), residual, fcw, fcb)
    return out[:, 0, :nc]


def _to_padded_flat(rows, N, T, H, W, C):
    Wp, Hp = W + 2, H + 2
    RS = _rs_of(H, W)
    x5 = rows.reshape(N, T, H, W, C)
    xp = jnp.pad(x5, ((0, 0), (1, 1), (1, 1), (1, 1), (0, 0)))
    flat = xp.reshape(N, T + 2, Hp * Wp, C)
    return jnp.pad(flat, ((0, 0), (0, 0), (0, RS - Hp * Wp), (0, 0)))


def _from_padded_flat(x_flat, N, T, H, W, C):
    Hp, Wp = H + 2, W + 2
    return x_flat[:, :, :Hp * Wp, :].reshape(N, T + 2, Hp, Wp, C)


def _im2col_strided(xp5, k, stride):
    N, Tp, Hp, Wp, C = xp5.shape
    oT = (Tp - k) // stride + 1
    oH = (Hp - k) // stride + 1
    oW = (Wp - k) // stride + 1
    patches = []
    for dt in range(k):
        for dh in range(k):
            for dw in range(k):
                patches.append(
                    xp5[:, dt:dt + (oT - 1) * stride + 1:stride,
                        dh:dh + (oH - 1) * stride + 1:stride,
                        dw:dw + (oW - 1) * stride + 1:stride, :])
    col = jnp.concatenate(patches, axis=-1)
    return col.reshape(N * oT * oH * oW, k * k * k * C), (N, oT, oH, oW)


def _strided_block0(x_flat, blk, dims_in, dims_out):
    _, oT, oH, oW = dims_out
    out1_flat, res_flat = _conv_s2(x_flat, blk, dims_in, dims_out)
    return _conv_s1(out1_flat, blk['conv2_w'], blk['bn2_scale'],
                    blk['bn2_shift'], H=oH, W=oW, residual=res_flat)


def kernel(x, stem_w, stem_scale, stem_shift,
           layer1_0_conv1_w, layer1_0_bn1_scale, layer1_0_bn1_shift,
           layer1_0_conv2_w, layer1_0_bn2_scale, layer1_0_bn2_shift,
           layer1_1_conv1_w, layer1_1_bn1_scale, layer1_1_bn1_shift,
           layer1_1_conv2_w, layer1_1_bn2_scale, layer1_1_bn2_shift,
           layer2_0_conv1_w, layer2_0_bn1_scale, layer2_0_bn1_shift,
           layer2_0_conv2_w, layer2_0_bn2_scale, layer2_0_bn2_shift,
           layer2_0_down_w, layer2_0_down_bn_scale, layer2_0_down_bn_shift,
           layer2_1_conv1_w, layer2_1_bn1_scale, layer2_1_bn1_shift,
           layer2_1_conv2_w, layer2_1_bn2_scale, layer2_1_bn2_shift,
           layer3_0_conv1_w, layer3_0_bn1_scale, layer3_0_bn1_shift,
           layer3_0_conv2_w, layer3_0_bn2_scale, layer3_0_bn2_shift,
           layer3_0_down_w, layer3_0_down_bn_scale, layer3_0_down_bn_shift,
           layer3_1_conv1_w, layer3_1_bn1_scale, layer3_1_bn1_shift,
           layer3_1_conv2_w, layer3_1_bn2_scale, layer3_1_bn2_shift,
           layer4_0_conv1_w, layer4_0_bn1_scale, layer4_0_bn1_shift,
           layer4_0_conv2_w, layer4_0_bn2_scale, layer4_0_bn2_shift,
           layer4_0_down_w, layer4_0_down_bn_scale, layer4_0_down_bn_shift,
           layer4_1_conv1_w, layer4_1_bn1_scale, layer4_1_bn1_shift,
           layer4_1_conv2_w, layer4_1_bn2_scale, layer4_1_bn2_shift,
           fc_w, fc_b):
    N = x.shape[0]
    oT, oH, oW = 8, 56, 56
    xp = jnp.pad(x.astype(jnp.bfloat16),
                 ((0, 0), (0, 0), (1, 1), (3, 3), (3, 3)))
    xq = xp.reshape(N, 3, 10, 59, 2, 59, 2)
    xq = jnp.transpose(xq, (0, 1, 4, 6, 2, 3, 5))
    patches = []
    for dt in range(3):
        for dh in range(7):
            for dw in range(7):
                for c in range(3):
                    patches.append(
                        xq[:, c, dh % 2, dw % 2, dt:dt + oT,
                           dh // 2:dh // 2 + oH, dw // 2:dw // 2 + oW])
    col_t = jnp.stack(patches).reshape(441, N * oT * oH * oW)
    w_mat = jnp.transpose(stem_w, (2, 3, 4, 1, 0)).reshape(441, 64)
    stem_out = _matmul_bn_ta(col_t, w_mat, stem_scale, stem_shift, relu=True)
    return stem_out[:4, :400].astype(jnp.float32)
    h = _to_padded_flat(stem_out, N, 8, 56, 56, 64)

    o1 = _conv_s1(h, layer1_0_conv1_w, layer1_0_bn1_scale, layer1_0_bn1_shift,
                  H=56, W=56)
    h = _conv_s1(o1, layer1_0_conv2_w, layer1_0_bn2_scale, layer1_0_bn2_shift,
                 H=56, W=56, residual=h)
    o1 = _conv_s1(h, layer1_1_conv1_w, layer1_1_bn1_scale, layer1_1_bn1_shift,
                  H=56, W=56)
    h = _conv_s1(o1, layer1_1_conv2_w, layer1_1_bn2_scale, layer1_1_bn2_shift,
                 H=56, W=56, residual=h)

    h = _strided_block0(
        h, {'conv1_w': layer2_0_conv1_w, 'bn1_scale': layer2_0_bn1_scale,
            'bn1_shift': layer2_0_bn1_shift, 'conv2_w': layer2_0_conv2_w,
            'bn2_scale': layer2_0_bn2_scale, 'bn2_shift': layer2_0_bn2_shift,
            'down_w': layer2_0_down_w, 'down_bn_scale': layer2_0_down_bn_scale,
            'down_bn_shift': layer2_0_down_bn_shift},
        (N, 8, 56, 56), (N, 4, 28, 28))
    o1 = _conv_s1(h, layer2_1_conv1_w, layer2_1_bn1_scale, layer2_1_bn1_shift,
                  H=28, W=28)
    h = _conv_s1(o1, layer2_1_conv2_w, layer2_1_bn2_scale, layer2_1_bn2_shift,
                 H=28, W=28, residual=h)

    h = _strided_block0(
        h, {'conv1_w': layer3_0_conv1_w, 'bn1_scale': layer3_0_bn1_scale,
            'bn1_shift': layer3_0_bn1_shift, 'conv2_w': layer3_0_conv2_w,
            'bn2_scale': layer3_0_bn2_scale, 'bn2_shift': layer3_0_bn2_shift,
            'down_w': layer3_0_down_w, 'down_bn_scale': layer3_0_down_bn_scale,
            'down_bn_shift': layer3_0_down_bn_shift},
        (N, 4, 28, 28), (N, 2, 14, 14))
    o1 = _conv_s1(h, layer3_1_conv1_w, layer3_1_bn1_scale, layer3_1_bn1_shift,
                  H=14, W=14)
    h = _conv_s1(o1, layer3_1_conv2_w, layer3_1_bn2_scale, layer3_1_bn2_shift,
                 H=14, W=14, residual=h)

    h = _strided_block0(
        h, {'conv1_w': layer4_0_conv1_w, 'bn1_scale': layer4_0_bn1_scale,
            'bn1_shift': layer4_0_bn1_shift, 'conv2_w': layer4_0_conv2_w,
            'bn2_scale': layer4_0_bn2_scale, 'bn2_shift': layer4_0_bn2_shift,
            'down_w': layer4_0_down_w, 'down_bn_scale': layer4_0_down_bn_scale,
            'down_bn_shift': layer4_0_down_bn_shift},
        (N, 2, 14, 14), (N, 1, 7, 7))
    o1 = _conv_s1(h, layer4_1_conv1_w, layer4_1_bn1_scale, layer4_1_bn1_shift,
                  H=7, W=7)
    logits = _conv_pool_fc(o1, layer4_1_conv2_w, layer4_1_bn2_scale,
                           layer4_1_bn2_shift, h, fc_w, fc_b, H=7, W=7)
    return logits


# speedup vs baseline: 4.4750x; 2.3655x over previous
import functools

import jax
import jax.numpy as jnp
from jax.experimental import pallas as pl
from jax.experimental.pallas import tpu as pltpu


def _rup(x, m):
    return (x + m - 1) // m * m


def _rs_of(H, W):
    base = (H + 2) * (W + 2)
    slack = 2 * (W + 2) + 2
    return _rup(base, 16) + slack


def _mmk(a_ref, b_ref, s_ref, t_ref, o_ref, *, relu):
    acc = jnp.dot(a_ref[...], b_ref[...], preferred_element_type=jnp.float32)
    y = acc * s_ref[...] + t_ref[...]
    if relu:
        y = jnp.maximum(y, 0.0)
    o_ref[...] = y.astype(o_ref.dtype)


def _matmul_bn(a, b, scale, shift, *, relu, tm=1024):
    M, K = a.shape
    N = b.shape[1]
    tm = min(tm, _rup(M, 8))
    Mp = _rup(M, tm)
    a = jnp.pad(a, ((0, Mp - M), (0, 0)))
    out = pl.pallas_call(
        functools.partial(_mmk, relu=relu),
        out_shape=jax.ShapeDtypeStruct((Mp, N), jnp.bfloat16),
        grid=(Mp // tm,),
        in_specs=[
            pl.BlockSpec((tm, K), lambda i: (i, 0)),
            pl.BlockSpec((K, N), lambda i: (0, 0)),
            pl.BlockSpec((1, N), lambda i: (0, 0)),
            pl.BlockSpec((1, N), lambda i: (0, 0)),
        ],
        out_specs=pl.BlockSpec((tm, N), lambda i: (i, 0)),
        compiler_params=pltpu.CompilerParams(
            dimension_semantics=("parallel",),
            vmem_limit_bytes=100 * 1024 * 1024),
    )(a, b.astype(jnp.bfloat16), scale.reshape(1, N).astype(jnp.float32),
      shift.reshape(1, N).astype(jnp.float32))
    return out[:M]


def _mmk_ta(a_ref, b_ref, s_ref, t_ref, o_ref, *, relu):
    acc = jax.lax.dot_general(a_ref[...], b_ref[...],
                              (((0,), (0,)), ((), ())),
                              preferred_element_type=jnp.float32)
    y = acc * s_ref[...] + t_ref[...]
    if relu:
        y = jnp.maximum(y, 0.0)
    o_ref[...] = y.astype(o_ref.dtype)


def _matmul_bn_ta(a_t, b, scale, shift, *, relu, tm=2048):
    K, M = a_t.shape
    N = b.shape[1]
    tm = min(tm, _rup(M, 128))
    Mp = _rup(M, tm)
    a_t = jnp.pad(a_t, ((0, 0), (0, Mp - M)))
    out = pl.pallas_call(
        functools.partial(_mmk_ta, relu=relu),
        out_shape=jax.ShapeDtypeStruct((Mp, N), jnp.bfloat16),
        grid=(Mp // tm,),
        in_specs=[
            pl.BlockSpec((K, tm), lambda i: (0, i)),
            pl.BlockSpec((K, N), lambda i: (0, 0)),
            pl.BlockSpec((1, N), lambda i: (0, 0)),
            pl.BlockSpec((1, N), lambda i: (0, 0)),
        ],
        out_specs=pl.BlockSpec((tm, N), lambda i: (i, 0)),
        compiler_params=pltpu.CompilerParams(
            dimension_semantics=("parallel",),
            vmem_limit_bytes=100 * 1024 * 1024),
    )(a_t, b.astype(jnp.bfloat16), scale.reshape(1, N).astype(jnp.float32),
      shift.reshape(1, N).astype(jnp.float32))
    return out[:M]


def _col_and_acc(x_refs, w_ref, *, M, Wp):
    pieces = []
    for x_ref in x_refs:
        xv = x_ref[0, 0]
        for dh in range(3):
            for dw in range(3):
                off = dh * Wp + dw
                pieces.append(xv[off:off + M, :])
    col = jnp.concatenate(pieces, axis=-1)
    return jnp.dot(col, w_ref[...], preferred_element_type=jnp.float32)


def _interior_mask(shape, *, H, W):
    Wp = W + 2
    r = jax.lax.broadcasted_iota(jnp.int32, shape, 0) + (Wp + 1)
    h = jnp.floor((r.astype(jnp.float32) + 0.5) *
                  jnp.float32(1.0 / Wp)).astype(jnp.int32)
    w = r - h * Wp
    return (h >= 1) & (h <= H) & (w >= 1) & (w <= W)


def _conv_s1_kernel(*refs, H, W, has_res, relu, Tp):
    if has_res:
        x0_ref, x1_ref, x2_ref, w_ref, s_ref, t_ref, res_ref, o_ref = refs
    else:
        x0_ref, x1_ref, x2_ref, w_ref, s_ref, t_ref, o_ref = refs
    tp = pl.program_id(1)
    Wp = W + 2
    RS = o_ref.shape[2]
    M = RS - (2 * Wp + 2)

    @pl.when(jnp.logical_or(tp == 0, tp == Tp - 1))
    def _():
        o_ref[...] = jnp.zeros_like(o_ref)

    @pl.when(jnp.logical_and(tp > 0, tp < Tp - 1))
    def _():
        D = Wp + 1
        acc = _col_and_acc((x0_ref, x1_ref, x2_ref), w_ref, M=M, Wp=Wp)
        y = acc * s_ref[...] + t_ref[...]
        if has_res:
            y = y + res_ref[0, 0, D:D + M, :].astype(jnp.float32)
        if relu:
            y = jnp.maximum(y, 0.0)
        y = jnp.where(_interior_mask(y.shape, H=H, W=W), y, 0.0)
        C = y.shape[1]
        o_ref[0, 0, 0:D, :] = jnp.zeros((D, C), o_ref.dtype)
        o_ref[0, 0, D:D + M, :] = y.astype(o_ref.dtype)
        o_ref[0, 0, D + M:RS, :] = jnp.zeros((RS - D - M, C), o_ref.dtype)


def _conv_s1(x, w, scale, shift, *, H, W, residual=None, relu=True):
    N, Tp, RS, Cin = x.shape
    Cout = w.shape[0]
    wm = jnp.transpose(w, (2, 3, 4, 1, 0)).reshape(27 * Cin, Cout)
    wm = wm.astype(jnp.bfloat16)
    sc = scale.reshape(1, Cout).astype(jnp.float32)
    sh = shift.reshape(1, Cout).astype(jnp.float32)
    in_specs = [
        pl.BlockSpec((1, 1, RS, Cin),
                     lambda n, t: (n, jnp.maximum(t - 1, 0), 0, 0)),
        pl.BlockSpec((1, 1, RS, Cin), lambda n, t: (n, t, 0, 0)),
        pl.BlockSpec((1, 1, RS, Cin),
                     lambda n, t: (n, jnp.minimum(t + 1, Tp - 1), 0, 0)),
        pl.BlockSpec((27 * Cin, Cout), lambda n, t: (0, 0)),
        pl.BlockSpec((1, Cout), lambda n, t: (0, 0)),
        pl.BlockSpec((1, Cout), lambda n, t: (0, 0)),
    ]
    args = [x, x, x, wm, sc, sh]
    if residual is not None:
        in_specs.append(pl.BlockSpec((1, 1, RS, Cout),
                                     lambda n, t: (n, t, 0, 0)))
        args.append(residual)
    kern = functools.partial(_conv_s1_kernel, H=H, W=W,
                             has_res=residual is not None, relu=relu, Tp=Tp)
    return pl.pallas_call(
        kern,
        out_shape=jax.ShapeDtypeStruct((N, Tp, RS, Cout), jnp.bfloat16),
        grid=(N, Tp),
        in_specs=in_specs,
        out_specs=pl.BlockSpec((1, 1, RS, Cout), lambda n, t: (n, t, 0, 0)),
        compiler_params=pltpu.CompilerParams(
            dimension_semantics=("parallel", "arbitrary"),
            vmem_limit_bytes=100 * 1024 * 1024),
    )(*args)


def _phase_split(x_flat, N, T, H, W, C, oH, oW):
    Hp, Wp = H + 2, W + 2
    Tpi = T + 2
    Wq = oW + 2
    M = _rup((oH + 2) * (oW + 2), 16)
    qmax = M + Wq + 2
    Hq = max((qmax + Wq - 1) // Wq + 1, (Hp + 1) // 2)
    RQ = _rup(Hq * Wq, 16)
    x5 = x_flat[:, :, :Hp * Wp, :].reshape(N, Tpi, Hp, Wp, C)
    x5 = jnp.pad(x5, ((0, 0), (0, 0), (0, 2 * Hq - Hp), (0, 2 * Wq - Wp),
                      (0, 0)))
    x5 = x5.reshape(N, Tpi, Hq, 2, Wq, 2, C)
    x5 = jnp.transpose(x5, (0, 1, 3, 5, 2, 4, 6))
    xq = x5.reshape(N, Tpi, 4, Hq * Wq, C)
    return jnp.pad(xq, ((0, 0), (0, 0), (0, 0), (0, RQ - Hq * Wq), (0, 0)))


def _conv_s2_kernel(x0_ref, x1_ref, x2_ref, w_ref, s_ref, t_ref,
                    wd_ref, ds_ref, dt_ref, o_ref, r_ref, *, oH, oW, Tpo):
    ts = pl.program_id(1)
    oWp = oW + 2
    RS = o_ref.shape[2]
    M = RS - (2 * oWp + 2)
    D = oWp + 1

    @pl.when(jnp.logical_or(ts == 0, ts == Tpo - 1))
    def _():
        o_ref[...] = jnp.zeros_like(o_ref)
        r_ref[...] = jnp.zeros_like(r_ref)

    @pl.when(jnp.logical_and(ts > 0, ts < Tpo - 1))
    def _():
        pieces = []
        for x_ref in (x0_ref, x1_ref, x2_ref):
            xv = x_ref[0, 0]
            for dh in range(3):
                for dw in range(3):
                    ph = (dh % 2) * 2 + (dw % 2)
                    off = (dh // 2) * oWp + (dw // 2)
                    pieces.append(xv[ph, off:off + M, :])
        col = jnp.concatenate(pieces, axis=-1)
        acc = jnp.dot(col, w_ref[...], preferred_element_type=jnp.float32)
        y = acc * s_ref[...] + t_ref[...]
        y = jnp.maximum(y, 0.0)
        mask = _interior_mask(y.shape, H=oH, W=oW)
        y = jnp.where(mask, y, 0.0)
        C = y.shape[1]
        o_ref[0, 0, 0:D, :] = jnp.zeros((D, C), o_ref.dtype)
        o_ref[0, 0, D:D + M, :] = y.astype(o_ref.dtype)
        o_ref[0, 0, D + M:RS, :] = jnp.zeros((RS - D - M, C), o_ref.dtype)
        accd = jnp.dot(x1_ref[0, 0, 3, 0:M, :], wd_ref[...],
                       preferred_element_type=jnp.float32)
        yr = accd * ds_ref[...] + dt_ref[...]
        yr = jnp.where(mask, yr, 0.0)
        r_ref[0, 0, 0:D, :] = jnp.zeros((D, C), r_ref.dtype)
        r_ref[0, 0, D:D + M, :] = yr.astype(r_ref.dtype)
        r_ref[0, 0, D + M:RS, :] = jnp.zeros((RS - D - M, C), r_ref.dtype)


def _conv_s2(x_flat, blk, dims_in, dims_out):
    N, T, H, W = dims_in
    _, oT, oH, oW = dims_out
    Cin = x_flat.shape[-1]
    Cout = blk['conv1_w'].shape[0]
    Tpi, Tpo = T + 2, oT + 2
    xq = _phase_split(x_flat, N, T, H, W, Cin, oH, oW)
    RQ = xq.shape[3]
    RSo = _rs_of(oH, oW)
    wm = jnp.transpose(blk['conv1_w'], (2, 3, 4, 1, 0)
                       ).reshape(27 * Cin, Cout).astype(jnp.bfloat16)
    wd = blk['down_w'].reshape(Cout, Cin).T.astype(jnp.bfloat16)
    sspec = pl.BlockSpec((1, Cout), lambda n, t: (0, 0))
    out1, res = pl.pallas_call(
        functools.partial(_conv_s2_kernel, oH=oH, oW=oW, Tpo=Tpo),
        out_shape=(jax.ShapeDtypeStruct((N, Tpo, RSo, Cout), jnp.bfloat16),
                   jax.ShapeDtypeStruct((N, Tpo, RSo, Cout), jnp.bfloat16)),
        grid=(N, Tpo),
        in_specs=[
            pl.BlockSpec((1, 1, 4, RQ, Cin),
                         lambda n, t: (n, jnp.clip(2 * t - 2, 0, Tpi - 1),
                                       0, 0, 0)),
            pl.BlockSpec((1, 1, 4, RQ, Cin),
                         lambda n, t: (n, jnp.clip(2 * t - 1, 0, Tpi - 1),
                                       0, 0, 0)),
            pl.BlockSpec((1, 1, 4, RQ, Cin),
                         lambda n, t: (n, jnp.clip(2 * t, 0, Tpi - 1),
                                       0, 0, 0)),
            pl.BlockSpec((27 * Cin, Cout), lambda n, t: (0, 0)),
            sspec, sspec,
            pl.BlockSpec((Cin, Cout), lambda n, t: (0, 0)),
            sspec, sspec,
        ],
        out_specs=(pl.BlockSpec((1, 1, RSo, Cout), lambda n, t: (n, t, 0, 0)),
                   pl.BlockSpec((1, 1, RSo, Cout),
                                lambda n, t: (n, t, 0, 0))),
        compiler_params=pltpu.CompilerParams(
            dimension_semantics=("parallel", "arbitrary"),
            vmem_limit_bytes=100 * 1024 * 1024),
    )(xq, xq, xq, wm,
      blk['bn1_scale'].reshape(1, Cout).astype(jnp.float32),
      blk['bn1_shift'].reshape(1, Cout).astype(jnp.float32),
      wd,
      blk['down_bn_scale'].reshape(1, Cout).astype(jnp.float32),
      blk['down_bn_shift'].reshape(1, Cout).astype(jnp.float32))
    return out1, res


def _stem_kernel(x0_ref, x1_ref, x2_ref, w_ref, s_ref, t_ref, o_ref, *, Tp):
    tp = pl.program_id(1)
    H = W = 56
    Wp = W + 2
    RS = o_ref.shape[2]
    M = RS - (2 * Wp + 2)
    D = Wp + 1

    @pl.when(jnp.logical_or(tp == 0, tp == Tp - 1))
    def _():
        o_ref[...] = jnp.zeros_like(o_ref)

    @pl.when(jnp.logical_and(tp > 0, tp < Tp - 1))
    def _():
        pieces = []
        for x_ref in (x0_ref, x1_ref, x2_ref):
            xv = x_ref[0, 0]
            for av in range(4):
                for au in range(4):
                    off = av * Wp + au
                    pieces.append(xv[off:off + M, :])
        col = jnp.concatenate(pieces, axis=-1)
        acc = jnp.dot(col, w_ref[...], preferred_element_type=jnp.float32)
        y = jnp.maximum(acc * s_ref[...] + t_ref[...], 0.0)
        y = jnp.where(_interior_mask(y.shape, H=H, W=W), y, 0.0)
        C = y.shape[1]
        o_ref[0, 0, 0:D, :] = jnp.zeros((D, C), o_ref.dtype)
        o_ref[0, 0, D:D + M, :] = y.astype(o_ref.dtype)
        o_ref[0, 0, D + M:RS, :] = jnp.zeros((RS - D - M, C), o_ref.dtype)


def _stem(x, stem_w, stem_scale, stem_shift):
    N = x.shape[0]
    RS1 = _rs_of(56, 56)
    M = RS1 - (2 * 58 + 2)
    xp = jnp.pad(x.astype(jnp.bfloat16),
                 ((0, 0), (0, 0), (1, 1), (3, 9), (3, 1)))
    xq = xp.reshape(N, 3, 10, 62, 2, 58, 2)
    xq = jnp.transpose(xq, (0, 2, 3, 5, 4, 6, 1))
    xq = jnp.pad(xq, ((0, 0),) * 6 + ((0, 1),))
    RQ = _rup(62 * 58, 16)
    x4 = jnp.pad(xq.reshape(N, 10, 62 * 58, 16),
                 ((0, 0), (0, 0), (0, RQ - 62 * 58), (0, 0)))
    wp = jnp.pad(stem_w, ((0, 0), (0, 1), (0, 0), (0, 1), (0, 1)))
    wp = wp.reshape(64, 4, 3, 4, 2, 4, 2)
    wp = jnp.transpose(wp, (2, 3, 5, 4, 6, 1, 0)).reshape(768, 64)
    Tp = 10
    return pl.pallas_call(
        functools.partial(_stem_kernel, Tp=Tp),
        out_shape=jax.ShapeDtypeStruct((N, Tp, RS1, 64), jnp.bfloat16),
        grid=(N, Tp),
        in_specs=[
            pl.BlockSpec((1, 1, RQ, 16),
                         lambda n, t: (n, jnp.maximum(t - 1, 0), 0, 0)),
            pl.BlockSpec((1, 1, RQ, 16), lambda n, t: (n, t, 0, 0)),
            pl.BlockSpec((1, 1, RQ, 16),
                         lambda n, t: (n, jnp.minimum(t + 1, Tp - 1), 0, 0)),
            pl.BlockSpec((768, 64), lambda n, t: (0, 0)),
            pl.BlockSpec((1, 64), lambda n, t: (0, 0)),
            pl.BlockSpec((1, 64), lambda n, t: (0, 0)),
        ],
        out_specs=pl.BlockSpec((1, 1, RS1, 64), lambda n, t: (n, t, 0, 0)),
        compiler_params=pltpu.CompilerParams(
            dimension_semantics=("parallel", "arbitrary"),
            vmem_limit_bytes=100 * 1024 * 1024),
    )(x4, x4, x4, wp.astype(jnp.bfloat16),
      stem_scale.reshape(1, 64).astype(jnp.float32),
      stem_shift.reshape(1, 64).astype(jnp.float32))


def _conv_pool_fc_kernel(x0_ref, x1_ref, x2_ref, w_ref, s_ref, t_ref,
                         res_ref, fcw_ref, fcb_ref, o_ref, *, H, W):
    Wp = W + 2
    RS = x1_ref.shape[2]
    M = RS - (2 * Wp + 2)
    D = Wp + 1
    acc = _col_and_acc((x0_ref, x1_ref, x2_ref), w_ref, M=M, Wp=Wp)
    y = acc * s_ref[...] + t_ref[...]
    y = y + res_ref[0, 0, D:D + M, :].astype(jnp.float32)
    y = jnp.maximum(y, 0.0)
    y = jnp.where(_interior_mask(y.shape, H=H, W=W), y, 0.0)
    pooled = jnp.sum(y, axis=0, keepdims=True) * (1.0 / (H * W))
    o_ref[0] = (jnp.dot(pooled, fcw_ref[...],
                        preferred_element_type=jnp.float32) + fcb_ref[...])


def _conv_pool_fc(x, w, scale, shift, residual, fc_w, fc_b, *, H, W):
    N, Tp, RS, Cin = x.shape
    Cout = w.shape[0]
    nc = fc_w.shape[1]
    NCp = _rup(nc, 128)
    wm = jnp.transpose(w, (2, 3, 4, 1, 0)).reshape(27 * Cin, Cout)
    wm = wm.astype(jnp.bfloat16)
    fcw = jnp.pad(fc_w.astype(jnp.float32), ((0, 0), (0, NCp - nc)))
    fcb = jnp.pad(fc_b.astype(jnp.float32), (0, NCp - nc)).reshape(1, NCp)
    out = pl.pallas_call(
        functools.partial(_conv_pool_fc_kernel, H=H, W=W),
        out_shape=jax.ShapeDtypeStruct((N, 1, NCp), jnp.float32),
        grid=(N,),
        in_specs=[
            pl.BlockSpec((1, 1, RS, Cin), lambda n: (n, 0, 0, 0)),
            pl.BlockSpec((1, 1, RS, Cin), lambda n: (n, 1, 0, 0)),
            pl.BlockSpec((1, 1, RS, Cin), lambda n: (n, 2, 0, 0)),
            pl.BlockSpec((27 * Cin, Cout), lambda n: (0, 0)),
            pl.BlockSpec((1, Cout), lambda n: (0, 0)),
            pl.BlockSpec((1, Cout), lambda n: (0, 0)),
            pl.BlockSpec((1, 1, RS, Cout), lambda n: (n, 1, 0, 0)),
            pl.BlockSpec((Cout, NCp), lambda n: (0, 0)),
            pl.BlockSpec((1, NCp), lambda n: (0, 0)),
        ],
        out_specs=pl.BlockSpec((1, 1, NCp), lambda n: (n, 0, 0)),
        compiler_params=pltpu.CompilerParams(
            dimension_semantics=("parallel",),
            vmem_limit_bytes=100 * 1024 * 1024),
    )(x, x, x, wm, scale.reshape(1, Cout).astype(jnp.float32),
      shift.reshape(1, Cout).astype(jnp.float32), residual, fcw, fcb)
    return out[:, 0, :nc]


def _to_padded_flat(rows, N, T, H, W, C):
    Wp, Hp = W + 2, H + 2
    RS = _rs_of(H, W)
    x5 = rows.reshape(N, T, H, W, C)
    xp = jnp.pad(x5, ((0, 0), (1, 1), (1, 1), (1, 1), (0, 0)))
    flat = xp.reshape(N, T + 2, Hp * Wp, C)
    return jnp.pad(flat, ((0, 0), (0, 0), (0, RS - Hp * Wp), (0, 0)))


def _from_padded_flat(x_flat, N, T, H, W, C):
    Hp, Wp = H + 2, W + 2
    return x_flat[:, :, :Hp * Wp, :].reshape(N, T + 2, Hp, Wp, C)


def _im2col_strided(xp5, k, stride):
    N, Tp, Hp, Wp, C = xp5.shape
    oT = (Tp - k) // stride + 1
    oH = (Hp - k) // stride + 1
    oW = (Wp - k) // stride + 1
    patches = []
    for dt in range(k):
        for dh in range(k):
            for dw in range(k):
                patches.append(
                    xp5[:, dt:dt + (oT - 1) * stride + 1:stride,
                        dh:dh + (oH - 1) * stride + 1:stride,
                        dw:dw + (oW - 1) * stride + 1:stride, :])
    col = jnp.concatenate(patches, axis=-1)
    return col.reshape(N * oT * oH * oW, k * k * k * C), (N, oT, oH, oW)


def _strided_block0(x_flat, blk, dims_in, dims_out):
    _, oT, oH, oW = dims_out
    out1_flat, res_flat = _conv_s2(x_flat, blk, dims_in, dims_out)
    return _conv_s1(out1_flat, blk['conv2_w'], blk['bn2_scale'],
                    blk['bn2_shift'], H=oH, W=oW, residual=res_flat)


def kernel(x, stem_w, stem_scale, stem_shift,
           layer1_0_conv1_w, layer1_0_bn1_scale, layer1_0_bn1_shift,
           layer1_0_conv2_w, layer1_0_bn2_scale, layer1_0_bn2_shift,
           layer1_1_conv1_w, layer1_1_bn1_scale, layer1_1_bn1_shift,
           layer1_1_conv2_w, layer1_1_bn2_scale, layer1_1_bn2_shift,
           layer2_0_conv1_w, layer2_0_bn1_scale, layer2_0_bn1_shift,
           layer2_0_conv2_w, layer2_0_bn2_scale, layer2_0_bn2_shift,
           layer2_0_down_w, layer2_0_down_bn_scale, layer2_0_down_bn_shift,
           layer2_1_conv1_w, layer2_1_bn1_scale, layer2_1_bn1_shift,
           layer2_1_conv2_w, layer2_1_bn2_scale, layer2_1_bn2_shift,
           layer3_0_conv1_w, layer3_0_bn1_scale, layer3_0_bn1_shift,
           layer3_0_conv2_w, layer3_0_bn2_scale, layer3_0_bn2_shift,
           layer3_0_down_w, layer3_0_down_bn_scale, layer3_0_down_bn_shift,
           layer3_1_conv1_w, layer3_1_bn1_scale, layer3_1_bn1_shift,
           layer3_1_conv2_w, layer3_1_bn2_scale, layer3_1_bn2_shift,
           layer4_0_conv1_w, layer4_0_bn1_scale, layer4_0_bn1_shift,
           layer4_0_conv2_w, layer4_0_bn2_scale, layer4_0_bn2_shift,
           layer4_0_down_w, layer4_0_down_bn_scale, layer4_0_down_bn_shift,
           layer4_1_conv1_w, layer4_1_bn1_scale, layer4_1_bn1_shift,
           layer4_1_conv2_w, layer4_1_bn2_scale, layer4_1_bn2_shift,
           fc_w, fc_b):
    N = x.shape[0]
    h = _stem(x, stem_w, stem_scale, stem_shift)

    o1 = _conv_s1(h, layer1_0_conv1_w, layer1_0_bn1_scale, layer1_0_bn1_shift,
                  H=56, W=56)
    h = _conv_s1(o1, layer1_0_conv2_w, layer1_0_bn2_scale, layer1_0_bn2_shift,
                 H=56, W=56, residual=h)
    o1 = _conv_s1(h, layer1_1_conv1_w, layer1_1_bn1_scale, layer1_1_bn1_shift,
                  H=56, W=56)
    h = _conv_s1(o1, layer1_1_conv2_w, layer1_1_bn2_scale, layer1_1_bn2_shift,
                 H=56, W=56, residual=h)

    h = _strided_block0(
        h, {'conv1_w': layer2_0_conv1_w, 'bn1_scale': layer2_0_bn1_scale,
            'bn1_shift': layer2_0_bn1_shift, 'conv2_w': layer2_0_conv2_w,
            'bn2_scale': layer2_0_bn2_scale, 'bn2_shift': layer2_0_bn2_shift,
            'down_w': layer2_0_down_w, 'down_bn_scale': layer2_0_down_bn_scale,
            'down_bn_shift': layer2_0_down_bn_shift},
        (N, 8, 56, 56), (N, 4, 28, 28))
    o1 = _conv_s1(h, layer2_1_conv1_w, layer2_1_bn1_scale, layer2_1_bn1_shift,
                  H=28, W=28)
    h = _conv_s1(o1, layer2_1_conv2_w, layer2_1_bn2_scale, layer2_1_bn2_shift,
                 H=28, W=28, residual=h)

    h = _strided_block0(
        h, {'conv1_w': layer3_0_conv1_w, 'bn1_scale': layer3_0_bn1_scale,
            'bn1_shift': layer3_0_bn1_shift, 'conv2_w': layer3_0_conv2_w,
            'bn2_scale': layer3_0_bn2_scale, 'bn2_shift': layer3_0_bn2_shift,
            'down_w': layer3_0_down_w, 'down_bn_scale': layer3_0_down_bn_scale,
            'down_bn_shift': layer3_0_down_bn_shift},
        (N, 4, 28, 28), (N, 2, 14, 14))
    o1 = _conv_s1(h, layer3_1_conv1_w, layer3_1_bn1_scale, layer3_1_bn1_shift,
                  H=14, W=14)
    h = _conv_s1(o1, layer3_1_conv2_w, layer3_1_bn2_scale, layer3_1_bn2_shift,
                 H=14, W=14, residual=h)

    h = _strided_block0(
        h, {'conv1_w': layer4_0_conv1_w, 'bn1_scale': layer4_0_bn1_scale,
            'bn1_shift': layer4_0_bn1_shift, 'conv2_w': layer4_0_conv2_w,
            'bn2_scale': layer4_0_bn2_scale, 'bn2_shift': layer4_0_bn2_shift,
            'down_w': layer4_0_down_w, 'down_bn_scale': layer4_0_down_bn_scale,
            'down_bn_shift': layer4_0_down_bn_shift},
        (N, 2, 14, 14), (N, 1, 7, 7))
    o1 = _conv_s1(h, layer4_1_conv1_w, layer4_1_bn1_scale, layer4_1_bn1_shift,
                  H=7, W=7)
    logits = _conv_pool_fc(o1, layer4_1_conv2_w, layer4_1_bn2_scale,
                           layer4_1_bn2_shift, h, fc_w, fc_b, H=7, W=7)
    return logits


# speedup vs baseline: 4.9990x; 1.1171x over previous
import functools

import jax
import jax.numpy as jnp
from jax.experimental import pallas as pl
from jax.experimental.pallas import tpu as pltpu


def _rup(x, m):
    return (x + m - 1) // m * m


def _rs_of(H, W):
    base = (H + 2) * (W + 2)
    slack = 2 * (W + 2) + 2
    return _rup(base, 16) + slack


def _mmk(a_ref, b_ref, s_ref, t_ref, o_ref, *, relu):
    acc = jnp.dot(a_ref[...], b_ref[...], preferred_element_type=jnp.float32)
    y = acc * s_ref[...] + t_ref[...]
    if relu:
        y = jnp.maximum(y, 0.0)
    o_ref[...] = y.astype(o_ref.dtype)


def _matmul_bn(a, b, scale, shift, *, relu, tm=1024):
    M, K = a.shape
    N = b.shape[1]
    tm = min(tm, _rup(M, 8))
    Mp = _rup(M, tm)
    a = jnp.pad(a, ((0, Mp - M), (0, 0)))
    out = pl.pallas_call(
        functools.partial(_mmk, relu=relu),
        out_shape=jax.ShapeDtypeStruct((Mp, N), jnp.bfloat16),
        grid=(Mp // tm,),
        in_specs=[
            pl.BlockSpec((tm, K), lambda i: (i, 0)),
            pl.BlockSpec((K, N), lambda i: (0, 0)),
            pl.BlockSpec((1, N), lambda i: (0, 0)),
            pl.BlockSpec((1, N), lambda i: (0, 0)),
        ],
        out_specs=pl.BlockSpec((tm, N), lambda i: (i, 0)),
        compiler_params=pltpu.CompilerParams(
            dimension_semantics=("parallel",),
            vmem_limit_bytes=100 * 1024 * 1024),
    )(a, b.astype(jnp.bfloat16), scale.reshape(1, N).astype(jnp.float32),
      shift.reshape(1, N).astype(jnp.float32))
    return out[:M]


def _mmk_ta(a_ref, b_ref, s_ref, t_ref, o_ref, *, relu):
    acc = jax.lax.dot_general(a_ref[...], b_ref[...],
                              (((0,), (0,)), ((), ())),
                              preferred_element_type=jnp.float32)
    y = acc * s_ref[...] + t_ref[...]
    if relu:
        y = jnp.maximum(y, 0.0)
    o_ref[...] = y.astype(o_ref.dtype)


def _matmul_bn_ta(a_t, b, scale, shift, *, relu, tm=2048):
    K, M = a_t.shape
    N = b.shape[1]
    tm = min(tm, _rup(M, 128))
    Mp = _rup(M, tm)
    a_t = jnp.pad(a_t, ((0, 0), (0, Mp - M)))
    out = pl.pallas_call(
        functools.partial(_mmk_ta, relu=relu),
        out_shape=jax.ShapeDtypeStruct((Mp, N), jnp.bfloat16),
        grid=(Mp // tm,),
        in_specs=[
            pl.BlockSpec((K, tm), lambda i: (0, i)),
            pl.BlockSpec((K, N), lambda i: (0, 0)),
            pl.BlockSpec((1, N), lambda i: (0, 0)),
            pl.BlockSpec((1, N), lambda i: (0, 0)),
        ],
        out_specs=pl.BlockSpec((tm, N), lambda i: (i, 0)),
        compiler_params=pltpu.CompilerParams(
            dimension_semantics=("parallel",),
            vmem_limit_bytes=100 * 1024 * 1024),
    )(a_t, b.astype(jnp.bfloat16), scale.reshape(1, N).astype(jnp.float32),
      shift.reshape(1, N).astype(jnp.float32))
    return out[:M]


def _col_and_acc(x_refs, w_ref, *, M, Wp):
    pieces = []
    for x_ref in x_refs:
        xv = x_ref[0, 0]
        for dh in range(3):
            for dw in range(3):
                off = dh * Wp + dw
                pieces.append(xv[off:off + M, :])
    col = jnp.concatenate(pieces, axis=-1)
    return jnp.dot(col, w_ref[...], preferred_element_type=jnp.float32)


def _interior_mask(shape, *, H, W):
    Wp = W + 2
    r = jax.lax.broadcasted_iota(jnp.int32, shape, 0) + (Wp + 1)
    h = jnp.floor((r.astype(jnp.float32) + 0.5) *
                  jnp.float32(1.0 / Wp)).astype(jnp.int32)
    w = r - h * Wp
    return (h >= 1) & (h <= H) & (w >= 1) & (w <= W)


def _conv_s1_kernel(*refs, H, W, has_res, relu, Tp):
    if has_res:
        x0_ref, x1_ref, x2_ref, w_ref, s_ref, t_ref, res_ref, o_ref = refs
    else:
        x0_ref, x1_ref, x2_ref, w_ref, s_ref, t_ref, o_ref = refs
    tp = pl.program_id(1)
    Wp = W + 2
    RS = o_ref.shape[2]
    M = RS - (2 * Wp + 2)

    @pl.when(jnp.logical_or(tp == 0, tp == Tp - 1))
    def _():
        o_ref[...] = jnp.zeros_like(o_ref)

    @pl.when(jnp.logical_and(tp > 0, tp < Tp - 1))
    def _():
        D = Wp + 1
        acc = _col_and_acc((x0_ref, x1_ref, x2_ref), w_ref, M=M, Wp=Wp)
        y = acc * s_ref[...] + t_ref[...]
        if has_res:
            y = y + res_ref[0, 0, D:D + M, :].astype(jnp.float32)
        if relu:
            y = jnp.maximum(y, 0.0)
        y = jnp.where(_interior_mask(y.shape, H=H, W=W), y, 0.0)
        C = y.shape[1]
        o_ref[0, 0, 0:D, :] = jnp.zeros((D, C), o_ref.dtype)
        o_ref[0, 0, D:D + M, :] = y.astype(o_ref.dtype)
        o_ref[0, 0, D + M:RS, :] = jnp.zeros((RS - D - M, C), o_ref.dtype)


def _conv_s1(x, w, scale, shift, *, H, W, residual=None, relu=True):
    N, Tp, RS, Cin = x.shape
    Cout = w.shape[0]
    wm = jnp.transpose(w.astype(jnp.bfloat16),
                       (2, 3, 4, 1, 0)).reshape(27 * Cin, Cout)
    sc = scale.reshape(1, Cout).astype(jnp.float32)
    sh = shift.reshape(1, Cout).astype(jnp.float32)
    in_specs = [
        pl.BlockSpec((1, 1, RS, Cin),
                     lambda n, t: (n, jnp.maximum(t - 1, 0), 0, 0)),
        pl.BlockSpec((1, 1, RS, Cin), lambda n, t: (n, t, 0, 0)),
        pl.BlockSpec((1, 1, RS, Cin),
                     lambda n, t: (n, jnp.minimum(t + 1, Tp - 1), 0, 0)),
        pl.BlockSpec((27 * Cin, Cout), lambda n, t: (0, 0)),
        pl.BlockSpec((1, Cout), lambda n, t: (0, 0)),
        pl.BlockSpec((1, Cout), lambda n, t: (0, 0)),
    ]
    args = [x, x, x, wm, sc, sh]
    if residual is not None:
        in_specs.append(pl.BlockSpec((1, 1, RS, Cout),
                                     lambda n, t: (n, t, 0, 0)))
        args.append(residual)
    kern = functools.partial(_conv_s1_kernel, H=H, W=W,
                             has_res=residual is not None, relu=relu, Tp=Tp)
    return pl.pallas_call(
        kern,
        out_shape=jax.ShapeDtypeStruct((N, Tp, RS, Cout), jnp.bfloat16),
        grid=(N, Tp),
        in_specs=in_specs,
        out_specs=pl.BlockSpec((1, 1, RS, Cout), lambda n, t: (n, t, 0, 0)),
        compiler_params=pltpu.CompilerParams(
            dimension_semantics=("parallel", "arbitrary"),
            vmem_limit_bytes=100 * 1024 * 1024),
    )(*args)


def _phase_split(x_flat, N, T, H, W, C, oH, oW):
    Hp, Wp = H + 2, W + 2
    Tpi = T + 2
    Wq = oW + 2
    M = _rup((oH + 2) * (oW + 2), 16)
    qmax = M + Wq + 2
    Hq = max((qmax + Wq - 1) // Wq + 1, (Hp + 1) // 2)
    RQ = _rup(Hq * Wq, 16)
    x5 = x_flat[:, :, :Hp * Wp, :].reshape(N, Tpi, Hp, Wp, C)
    x5 = jnp.pad(x5, ((0, 0), (0, 0), (0, 2 * Hq - Hp), (0, 2 * Wq - Wp),
                      (0, 0)))
    x5 = x5.reshape(N, Tpi, Hq, 2, Wq, 2, C)
    x5 = jnp.transpose(x5, (0, 1, 3, 5, 2, 4, 6))
    xq = x5.reshape(N, Tpi, 4, Hq * Wq, C)
    return jnp.pad(xq, ((0, 0), (0, 0), (0, 0), (0, RQ - Hq * Wq), (0, 0)))


def _conv_s2_kernel(x0_ref, x1_ref, x2_ref, w_ref, s_ref, t_ref,
                    wd_ref, ds_ref, dt_ref, o_ref, r_ref, *, oH, oW, Tpo):
    ts = pl.program_id(1)
    oWp = oW + 2
    RS = o_ref.shape[2]
    M = RS - (2 * oWp + 2)
    D = oWp + 1

    @pl.when(jnp.logical_or(ts == 0, ts == Tpo - 1))
    def _():
        o_ref[...] = jnp.zeros_like(o_ref)
        r_ref[...] = jnp.zeros_like(r_ref)

    @pl.when(jnp.logical_and(ts > 0, ts < Tpo - 1))
    def _():
        pieces = []
        for x_ref in (x0_ref, x1_ref, x2_ref):
            xv = x_ref[0, 0]
            for dh in range(3):
                for dw in range(3):
                    ph = (dh % 2) * 2 + (dw % 2)
                    off = (dh // 2) * oWp + (dw // 2)
                    pieces.append(xv[ph, off:off + M, :])
        col = jnp.concatenate(pieces, axis=-1)
        acc = jnp.dot(col, w_ref[...], preferred_element_type=jnp.float32)
        y = acc * s_ref[...] + t_ref[...]
        y = jnp.maximum(y, 0.0)
        mask = _interior_mask(y.shape, H=oH, W=oW)
        y = jnp.where(mask, y, 0.0)
        C = y.shape[1]
        o_ref[0, 0, 0:D, :] = jnp.zeros((D, C), o_ref.dtype)
        o_ref[0, 0, D:D + M, :] = y.astype(o_ref.dtype)
        o_ref[0, 0, D + M:RS, :] = jnp.zeros((RS - D - M, C), o_ref.dtype)
        accd = jnp.dot(x1_ref[0, 0, 3, 0:M, :], wd_ref[...],
                       preferred_element_type=jnp.float32)
        yr = accd * ds_ref[...] + dt_ref[...]
        yr = jnp.where(mask, yr, 0.0)
        r_ref[0, 0, 0:D, :] = jnp.zeros((D, C), r_ref.dtype)
        r_ref[0, 0, D:D + M, :] = yr.astype(r_ref.dtype)
        r_ref[0, 0, D + M:RS, :] = jnp.zeros((RS - D - M, C), r_ref.dtype)


def _conv_s2(x_flat, blk, dims_in, dims_out):
    N, T, H, W = dims_in
    _, oT, oH, oW = dims_out
    Cin = x_flat.shape[-1]
    Cout = blk['conv1_w'].shape[0]
    Tpi, Tpo = T + 2, oT + 2
    xq = _phase_split(x_flat, N, T, H, W, Cin, oH, oW)
    RQ = xq.shape[3]
    RSo = _rs_of(oH, oW)
    wm = jnp.transpose(blk['conv1_w'].astype(jnp.bfloat16),
                       (2, 3, 4, 1, 0)).reshape(27 * Cin, Cout)
    wd = blk['down_w'].reshape(Cout, Cin).T.astype(jnp.bfloat16)
    sspec = pl.BlockSpec((1, Cout), lambda n, t: (0, 0))
    out1, res = pl.pallas_call(
        functools.partial(_conv_s2_kernel, oH=oH, oW=oW, Tpo=Tpo),
        out_shape=(jax.ShapeDtypeStruct((N, Tpo, RSo, Cout), jnp.bfloat16),
                   jax.ShapeDtypeStruct((N, Tpo, RSo, Cout), jnp.bfloat16)),
        grid=(N, Tpo),
        in_specs=[
            pl.BlockSpec((1, 1, 4, RQ, Cin),
                         lambda n, t: (n, jnp.clip(2 * t - 2, 0, Tpi - 1),
                                       0, 0, 0)),
            pl.BlockSpec((1, 1, 4, RQ, Cin),
                         lambda n, t: (n, jnp.clip(2 * t - 1, 0, Tpi - 1),
                                       0, 0, 0)),
            pl.BlockSpec((1, 1, 4, RQ, Cin),
                         lambda n, t: (n, jnp.clip(2 * t, 0, Tpi - 1),
                                       0, 0, 0)),
            pl.BlockSpec((27 * Cin, Cout), lambda n, t: (0, 0)),
            sspec, sspec,
            pl.BlockSpec((Cin, Cout), lambda n, t: (0, 0)),
            sspec, sspec,
        ],
        out_specs=(pl.BlockSpec((1, 1, RSo, Cout), lambda n, t: (n, t, 0, 0)),
                   pl.BlockSpec((1, 1, RSo, Cout),
                                lambda n, t: (n, t, 0, 0))),
        compiler_params=pltpu.CompilerParams(
            dimension_semantics=("parallel", "arbitrary"),
            vmem_limit_bytes=100 * 1024 * 1024),
    )(xq, xq, xq, wm,
      blk['bn1_scale'].reshape(1, Cout).astype(jnp.float32),
      blk['bn1_shift'].reshape(1, Cout).astype(jnp.float32),
      wd,
      blk['down_bn_scale'].reshape(1, Cout).astype(jnp.float32),
      blk['down_bn_shift'].reshape(1, Cout).astype(jnp.float32))
    return out1, res


def _stem_kernel(x0_ref, x1_ref, x2_ref, w_ref, s_ref, t_ref, o_ref, *, Tp):
    tp = pl.program_id(1)
    H = W = 56
    Wp = W + 2
    RS = o_ref.shape[2]
    M = RS - (2 * Wp + 2)
    D = Wp + 1

    @pl.when(jnp.logical_or(tp == 0, tp == Tp - 1))
    def _():
        o_ref[...] = jnp.zeros_like(o_ref)

    @pl.when(jnp.logical_and(tp > 0, tp < Tp - 1))
    def _():
        pieces = []
        for x_ref in (x0_ref, x1_ref, x2_ref):
            xv = x_ref[0, 0]
            for av in range(4):
                pieces.append(xv[av * Wp:av * Wp + M, :])
        col = jnp.concatenate(pieces, axis=-1)
        acc = jnp.dot(col, w_ref[...], preferred_element_type=jnp.float32)
        y = jnp.maximum(acc * s_ref[...] + t_ref[...], 0.0)
        y = jnp.where(_interior_mask(y.shape, H=H, W=W), y, 0.0)
        C = y.shape[1]
        o_ref[0, 0, 0:D, :] = jnp.zeros((D, C), o_ref.dtype)
        o_ref[0, 0, D:D + M, :] = y.astype(o_ref.dtype)
        o_ref[0, 0, D + M:RS, :] = jnp.zeros((RS - D - M, C), o_ref.dtype)


def _stem(x, stem_w, stem_scale, stem_shift):
    N = x.shape[0]
    RS1 = _rs_of(56, 56)
    M = RS1 - (2 * 58 + 2)
    xp = jnp.pad(x.astype(jnp.bfloat16),
                 ((0, 0), (0, 0), (1, 1), (3, 9), (3, 9)))
    xq = xp.reshape(N, 3, 10, 62, 2, 62, 2)
    xq = jnp.transpose(xq, (0, 2, 3, 5, 4, 6, 1))
    xq = jnp.pad(xq, ((0, 0),) * 6 + ((0, 1),))
    xq = xq.reshape(N, 10, 62, 62, 16)
    x4 = jnp.concatenate([xq[:, :, :, au:au + 58] for au in range(4)],
                         axis=-1)
    RQ = _rup(62 * 58, 16)
    x4 = jnp.pad(x4.reshape(N, 10, 62 * 58, 64),
                 ((0, 0), (0, 0), (0, RQ - 62 * 58), (0, 0)))
    wp = jnp.pad(stem_w, ((0, 0), (0, 1), (0, 0), (0, 1), (0, 1)))
    wp = wp.reshape(64, 4, 3, 4, 2, 4, 2)
    wp = jnp.transpose(wp, (2, 3, 5, 4, 6, 1, 0)).reshape(768, 64)
    Tp = 10
    return pl.pallas_call(
        functools.partial(_stem_kernel, Tp=Tp),
        out_shape=jax.ShapeDtypeStruct((N, Tp, RS1, 64), jnp.bfloat16),
        grid=(N, Tp),
        in_specs=[
            pl.BlockSpec((1, 1, RQ, 64),
                         lambda n, t: (n, jnp.maximum(t - 1, 0), 0, 0)),
            pl.BlockSpec((1, 1, RQ, 64), lambda n, t: (n, t, 0, 0)),
            pl.BlockSpec((1, 1, RQ, 64),
                         lambda n, t: (n, jnp.minimum(t + 1, Tp - 1), 0, 0)),
            pl.BlockSpec((768, 64), lambda n, t: (0, 0)),
            pl.BlockSpec((1, 64), lambda n, t: (0, 0)),
            pl.BlockSpec((1, 64), lambda n, t: (0, 0)),
        ],
        out_specs=pl.BlockSpec((1, 1, RS1, 64), lambda n, t: (n, t, 0, 0)),
        compiler_params=pltpu.CompilerParams(
            dimension_semantics=("parallel", "arbitrary"),
            vmem_limit_bytes=100 * 1024 * 1024),
    )(x4, x4, x4, wp.astype(jnp.bfloat16),
      stem_scale.reshape(1, 64).astype(jnp.float32),
      stem_shift.reshape(1, 64).astype(jnp.float32))


def _conv_pool_fc_kernel(x0_ref, x1_ref, x2_ref, w_ref, s_ref, t_ref,
                         res_ref, fcw_ref, fcb_ref, o_ref, *, H, W):
    Wp = W + 2
    RS = x1_ref.shape[2]
    M = RS - (2 * Wp + 2)
    D = Wp + 1
    acc = _col_and_acc((x0_ref, x1_ref, x2_ref), w_ref, M=M, Wp=Wp)
    y = acc * s_ref[...] + t_ref[...]
    y = y + res_ref[0, 0, D:D + M, :].astype(jnp.float32)
    y = jnp.maximum(y, 0.0)
    y = jnp.where(_interior_mask(y.shape, H=H, W=W), y, 0.0)
    pooled = jnp.sum(y, axis=0, keepdims=True) * (1.0 / (H * W))
    o_ref[0] = (jnp.dot(pooled, fcw_ref[...],
                        preferred_element_type=jnp.float32) + fcb_ref[...])


def _conv_pool_fc(x, w, scale, shift, residual, fc_w, fc_b, *, H, W):
    N, Tp, RS, Cin = x.shape
    Cout = w.shape[0]
    nc = fc_w.shape[1]
    NCp = _rup(nc, 128)
    wm = jnp.transpose(w.astype(jnp.bfloat16),
                       (2, 3, 4, 1, 0)).reshape(27 * Cin, Cout)
    fcw = jnp.pad(fc_w.astype(jnp.float32), ((0, 0), (0, NCp - nc)))
    fcb = jnp.pad(fc_b.astype(jnp.float32), (0, NCp - nc)).reshape(1, NCp)
    out = pl.pallas_call(
        functools.partial(_conv_pool_fc_kernel, H=H, W=W),
        out_shape=jax.ShapeDtypeStruct((N, 1, NCp), jnp.float32),
        grid=(N,),
        in_specs=[
            pl.BlockSpec((1, 1, RS, Cin), lambda n: (n, 0, 0, 0)),
            pl.BlockSpec((1, 1, RS, Cin), lambda n: (n, 1, 0, 0)),
            pl.BlockSpec((1, 1, RS, Cin), lambda n: (n, 2, 0, 0)),
            pl.BlockSpec((27 * Cin, Cout), lambda n: (0, 0)),
            pl.BlockSpec((1, Cout), lambda n: (0, 0)),
            pl.BlockSpec((1, Cout), lambda n: (0, 0)),
            pl.BlockSpec((1, 1, RS, Cout), lambda n: (n, 1, 0, 0)),
            pl.BlockSpec((Cout, NCp), lambda n: (0, 0)),
            pl.BlockSpec((1, NCp), lambda n: (0, 0)),
        ],
        out_specs=pl.BlockSpec((1, 1, NCp), lambda n: (n, 0, 0)),
        compiler_params=pltpu.CompilerParams(
            dimension_semantics=("parallel",),
            vmem_limit_bytes=100 * 1024 * 1024),
    )(x, x, x, wm, scale.reshape(1, Cout).astype(jnp.float32),
      shift.reshape(1, Cout).astype(jnp.float32), residual, fcw, fcb)
    return out[:, 0, :nc]


def _to_padded_flat(rows, N, T, H, W, C):
    Wp, Hp = W + 2, H + 2
    RS = _rs_of(H, W)
    x5 = rows.reshape(N, T, H, W, C)
    xp = jnp.pad(x5, ((0, 0), (1, 1), (1, 1), (1, 1), (0, 0)))
    flat = xp.reshape(N, T + 2, Hp * Wp, C)
    return jnp.pad(flat, ((0, 0), (0, 0), (0, RS - Hp * Wp), (0, 0)))


def _from_padded_flat(x_flat, N, T, H, W, C):
    Hp, Wp = H + 2, W + 2
    return x_flat[:, :, :Hp * Wp, :].reshape(N, T + 2, Hp, Wp, C)


def _im2col_strided(xp5, k, stride):
    N, Tp, Hp, Wp, C = xp5.shape
    oT = (Tp - k) // stride + 1
    oH = (Hp - k) // stride + 1
    oW = (Wp - k) // stride + 1
    patches = []
    for dt in range(k):
        for dh in range(k):
            for dw in range(k):
                patches.append(
                    xp5[:, dt:dt + (oT - 1) * stride + 1:stride,
                        dh:dh + (oH - 1) * stride + 1:stride,
                        dw:dw + (oW - 1) * stride + 1:stride, :])
    col = jnp.concatenate(patches, axis=-1)
    return col.reshape(N * oT * oH * oW, k * k * k * C), (N, oT, oH, oW)


def _strided_block0(x_flat, blk, dims_in, dims_out):
    _, oT, oH, oW = dims_out
    out1_flat, res_flat = _conv_s2(x_flat, blk, dims_in, dims_out)
    return _conv_s1(out1_flat, blk['conv2_w'], blk['bn2_scale'],
                    blk['bn2_shift'], H=oH, W=oW, residual=res_flat)


def kernel(x, stem_w, stem_scale, stem_shift,
           layer1_0_conv1_w, layer1_0_bn1_scale, layer1_0_bn1_shift,
           layer1_0_conv2_w, layer1_0_bn2_scale, layer1_0_bn2_shift,
           layer1_1_conv1_w, layer1_1_bn1_scale, layer1_1_bn1_shift,
           layer1_1_conv2_w, layer1_1_bn2_scale, layer1_1_bn2_shift,
           layer2_0_conv1_w, layer2_0_bn1_scale, layer2_0_bn1_shift,
           layer2_0_conv2_w, layer2_0_bn2_scale, layer2_0_bn2_shift,
           layer2_0_down_w, layer2_0_down_bn_scale, layer2_0_down_bn_shift,
           layer2_1_conv1_w, layer2_1_bn1_scale, layer2_1_bn1_shift,
           layer2_1_conv2_w, layer2_1_bn2_scale, layer2_1_bn2_shift,
           layer3_0_conv1_w, layer3_0_bn1_scale, layer3_0_bn1_shift,
           layer3_0_conv2_w, layer3_0_bn2_scale, layer3_0_bn2_shift,
           layer3_0_down_w, layer3_0_down_bn_scale, layer3_0_down_bn_shift,
           layer3_1_conv1_w, layer3_1_bn1_scale, layer3_1_bn1_shift,
           layer3_1_conv2_w, layer3_1_bn2_scale, layer3_1_bn2_shift,
           layer4_0_conv1_w, layer4_0_bn1_scale, layer4_0_bn1_shift,
           layer4_0_conv2_w, layer4_0_bn2_scale, layer4_0_bn2_shift,
           layer4_0_down_w, layer4_0_down_bn_scale, layer4_0_down_bn_shift,
           layer4_1_conv1_w, layer4_1_bn1_scale, layer4_1_bn1_shift,
           layer4_1_conv2_w, layer4_1_bn2_scale, layer4_1_bn2_shift,
           fc_w, fc_b):
    N = x.shape[0]
    h = _stem(x, stem_w, stem_scale, stem_shift)

    o1 = _conv_s1(h, layer1_0_conv1_w, layer1_0_bn1_scale, layer1_0_bn1_shift,
                  H=56, W=56)
    h = _conv_s1(o1, layer1_0_conv2_w, layer1_0_bn2_scale, layer1_0_bn2_shift,
                 H=56, W=56, residual=h)
    o1 = _conv_s1(h, layer1_1_conv1_w, layer1_1_bn1_scale, layer1_1_bn1_shift,
                  H=56, W=56)
    h = _conv_s1(o1, layer1_1_conv2_w, layer1_1_bn2_scale, layer1_1_bn2_shift,
                 H=56, W=56, residual=h)

    h = _strided_block0(
        h, {'conv1_w': layer2_0_conv1_w, 'bn1_scale': layer2_0_bn1_scale,
            'bn1_shift': layer2_0_bn1_shift, 'conv2_w': layer2_0_conv2_w,
            'bn2_scale': layer2_0_bn2_scale, 'bn2_shift': layer2_0_bn2_shift,
            'down_w': layer2_0_down_w, 'down_bn_scale': layer2_0_down_bn_scale,
            'down_bn_shift': layer2_0_down_bn_shift},
        (N, 8, 56, 56), (N, 4, 28, 28))
    o1 = _conv_s1(h, layer2_1_conv1_w, layer2_1_bn1_scale, layer2_1_bn1_shift,
                  H=28, W=28)
    h = _conv_s1(o1, layer2_1_conv2_w, layer2_1_bn2_scale, layer2_1_bn2_shift,
                 H=28, W=28, residual=h)

    h = _strided_block0(
        h, {'conv1_w': layer3_0_conv1_w, 'bn1_scale': layer3_0_bn1_scale,
            'bn1_shift': layer3_0_bn1_shift, 'conv2_w': layer3_0_conv2_w,
            'bn2_scale': layer3_0_bn2_scale, 'bn2_shift': layer3_0_bn2_shift,
            'down_w': layer3_0_down_w, 'down_bn_scale': layer3_0_down_bn_scale,
            'down_bn_shift': layer3_0_down_bn_shift},
        (N, 4, 28, 28), (N, 2, 14, 14))
    o1 = _conv_s1(h, layer3_1_conv1_w, layer3_1_bn1_scale, layer3_1_bn1_shift,
                  H=14, W=14)
    h = _conv_s1(o1, layer3_1_conv2_w, layer3_1_bn2_scale, layer3_1_bn2_shift,
                 H=14, W=14, residual=h)

    h = _strided_block0(
        h, {'conv1_w': layer4_0_conv1_w, 'bn1_scale': layer4_0_bn1_scale,
            'bn1_shift': layer4_0_bn1_shift, 'conv2_w': layer4_0_conv2_w,
            'bn2_scale': layer4_0_bn2_scale, 'bn2_shift': layer4_0_bn2_shift,
            'down_w': layer4_0_down_w, 'down_bn_scale': layer4_0_down_bn_scale,
            'down_bn_shift': layer4_0_down_bn_shift},
        (N, 2, 14, 14), (N, 1, 7, 7))
    o1 = _conv_s1(h, layer4_1_conv1_w, layer4_1_bn1_scale, layer4_1_bn1_shift,
                  H=7, W=7)
    logits = _conv_pool_fc(o1, layer4_1_conv2_w, layer4_1_bn2_scale,
                           layer4_1_bn2_shift, h, fc_w, fc_b, H=7, W=7)
    return logits


# speedup vs baseline: 5.0051x; 1.0012x over previous
import functools

import jax
import jax.numpy as jnp
from jax.experimental import pallas as pl
from jax.experimental.pallas import tpu as pltpu


def _rup(x, m):
    return (x + m - 1) // m * m


def _rs_of(H, W):
    base = (H + 2) * (W + 2)
    slack = 2 * (W + 2) + 2
    return _rup(base, 16) + slack


def _col_and_acc(x_refs, w_ref, *, M, Wp):
    pieces = []
    for x_ref in x_refs:
        xv = x_ref[0, 0]
        for dh in range(3):
            for dw in range(3):
                off = dh * Wp + dw
                pieces.append(xv[off:off + M, :])
    col = jnp.concatenate(pieces, axis=-1)
    return jnp.dot(col, w_ref[...], preferred_element_type=jnp.float32)


def _interior_mask(shape, *, H, W):
    Wp = W + 2
    r = jax.lax.broadcasted_iota(jnp.int32, shape, 0) + (Wp + 1)
    h = jnp.floor((r.astype(jnp.float32) + 0.5) *
                  jnp.float32(1.0 / Wp)).astype(jnp.int32)
    w = r - h * Wp
    return (h >= 1) & (h <= H) & (w >= 1) & (w <= W)


def _conv_s1_kernel(*refs, H, W, has_res, relu, Tp):
    if has_res:
        x0_ref, x1_ref, x2_ref, w_ref, s_ref, t_ref, res_ref, o_ref = refs
    else:
        x0_ref, x1_ref, x2_ref, w_ref, s_ref, t_ref, o_ref = refs
    tp = pl.program_id(1)
    Wp = W + 2
    RS = o_ref.shape[2]
    M = RS - (2 * Wp + 2)

    @pl.when(jnp.logical_or(tp == 0, tp == Tp - 1))
    def _():
        o_ref[...] = jnp.zeros_like(o_ref)

    @pl.when(jnp.logical_and(tp > 0, tp < Tp - 1))
    def _():
        D = Wp + 1
        acc = _col_and_acc((x0_ref, x1_ref, x2_ref), w_ref, M=M, Wp=Wp)
        y = acc * s_ref[...] + t_ref[...]
        if has_res:
            y = y + res_ref[0, 0, D:D + M, :].astype(jnp.float32)
        if relu:
            y = jnp.maximum(y, 0.0)
        y = jnp.where(_interior_mask(y.shape, H=H, W=W), y, 0.0)
        C = y.shape[1]
        o_ref[0, 0, 0:D, :] = jnp.zeros((D, C), o_ref.dtype)
        o_ref[0, 0, D:D + M, :] = y.astype(o_ref.dtype)
        o_ref[0, 0, D + M:RS, :] = jnp.zeros((RS - D - M, C), o_ref.dtype)


def _conv_s1(x, w, scale, shift, *, H, W, residual=None, relu=True):
    N, Tp, RS, Cin = x.shape
    Cout = w.shape[0]
    wm = jnp.transpose(w.astype(jnp.bfloat16),
                       (2, 3, 4, 1, 0)).reshape(27 * Cin, Cout)
    sc = scale.reshape(1, Cout).astype(jnp.float32)
    sh = shift.reshape(1, Cout).astype(jnp.float32)
    in_specs = [
        pl.BlockSpec((1, 1, RS, Cin),
                     lambda n, t: (n, jnp.maximum(t - 1, 0), 0, 0)),
        pl.BlockSpec((1, 1, RS, Cin), lambda n, t: (n, t, 0, 0)),
        pl.BlockSpec((1, 1, RS, Cin),
                     lambda n, t: (n, jnp.minimum(t + 1, Tp - 1), 0, 0)),
        pl.BlockSpec((27 * Cin, Cout), lambda n, t: (0, 0)),
        pl.BlockSpec((1, Cout), lambda n, t: (0, 0)),
        pl.BlockSpec((1, Cout), lambda n, t: (0, 0)),
    ]
    args = [x, x, x, wm, sc, sh]
    if residual is not None:
        in_specs.append(pl.BlockSpec((1, 1, RS, Cout),
                                     lambda n, t: (n, t, 0, 0)))
        args.append(residual)
    kern = functools.partial(_conv_s1_kernel, H=H, W=W,
                             has_res=residual is not None, relu=relu, Tp=Tp)
    return pl.pallas_call(
        kern,
        out_shape=jax.ShapeDtypeStruct((N, Tp, RS, Cout), jnp.bfloat16),
        grid=(N, Tp),
        in_specs=in_specs,
        out_specs=pl.BlockSpec((1, 1, RS, Cout), lambda n, t: (n, t, 0, 0)),
        compiler_params=pltpu.CompilerParams(
            dimension_semantics=("parallel", "arbitrary"),
            vmem_limit_bytes=100 * 1024 * 1024),
    )(*args)


def _phase_split(x_flat, N, T, H, W, C, oH, oW):
    Hp, Wp = H + 2, W + 2
    Tpi = T + 2
    Wq = oW + 2
    M = _rup((oH + 2) * (oW + 2), 16)
    qmax = M + Wq + 2
    Hq = max((qmax + Wq - 1) // Wq + 1, (Hp + 1) // 2)
    RQ = _rup(Hq * Wq, 16)
    x5 = x_flat[:, :, :Hp * Wp, :].reshape(N, Tpi, Hp, Wp, C)
    x5 = jnp.pad(x5, ((0, 0), (0, 0), (0, 2 * Hq - Hp), (0, 2 * Wq - Wp),
                      (0, 0)))
    x5 = x5.reshape(N, Tpi, Hq, 2, Wq, 2, C)
    x5 = jnp.transpose(x5, (0, 1, 3, 5, 2, 4, 6))
    xq = x5.reshape(N, Tpi, 4, Hq * Wq, C)
    return jnp.pad(xq, ((0, 0), (0, 0), (0, 0), (0, RQ - Hq * Wq), (0, 0)))


def _conv_s2_kernel(x0_ref, x1_ref, x2_ref, w_ref, s_ref, t_ref,
                    wd_ref, ds_ref, dt_ref, o_ref, r_ref, *, oH, oW, Tpo):
    ts = pl.program_id(1)
    oWp = oW + 2
    RS = o_ref.shape[2]
    M = RS - (2 * oWp + 2)
    D = oWp + 1

    @pl.when(jnp.logical_or(ts == 0, ts == Tpo - 1))
    def _():
        o_ref[...] = jnp.zeros_like(o_ref)
        r_ref[...] = jnp.zeros_like(r_ref)

    @pl.when(jnp.logical_and(ts > 0, ts < Tpo - 1))
    def _():
        pieces = []
        for x_ref in (x0_ref, x1_ref, x2_ref):
            xv = x_ref[0, 0]
            for dh in range(3):
                for dw in range(3):
                    ph = (dh % 2) * 2 + (dw % 2)
                    off = (dh // 2) * oWp + (dw // 2)
                    pieces.append(xv[ph, off:off + M, :])
        col = jnp.concatenate(pieces, axis=-1)
        acc = jnp.dot(col, w_ref[...], preferred_element_type=jnp.float32)
        y = acc * s_ref[...] + t_ref[...]
        y = jnp.maximum(y, 0.0)
        mask = _interior_mask(y.shape, H=oH, W=oW)
        y = jnp.where(mask, y, 0.0)
        C = y.shape[1]
        o_ref[0, 0, 0:D, :] = jnp.zeros((D, C), o_ref.dtype)
        o_ref[0, 0, D:D + M, :] = y.astype(o_ref.dtype)
        o_ref[0, 0, D + M:RS, :] = jnp.zeros((RS - D - M, C), o_ref.dtype)
        accd = jnp.dot(x1_ref[0, 0, 3, 0:M, :], wd_ref[...],
                       preferred_element_type=jnp.float32)
        yr = accd * ds_ref[...] + dt_ref[...]
        yr = jnp.where(mask, yr, 0.0)
        r_ref[0, 0, 0:D, :] = jnp.zeros((D, C), r_ref.dtype)
        r_ref[0, 0, D:D + M, :] = yr.astype(r_ref.dtype)
        r_ref[0, 0, D + M:RS, :] = jnp.zeros((RS - D - M, C), r_ref.dtype)


def _conv_s2(x_flat, blk, dims_in, dims_out):
    N, T, H, W = dims_in
    _, oT, oH, oW = dims_out
    Cin = x_flat.shape[-1]
    Cout = blk['conv1_w'].shape[0]
    Tpi, Tpo = T + 2, oT + 2
    xq = _phase_split(x_flat, N, T, H, W, Cin, oH, oW)
    RQ = xq.shape[3]
    RSo = _rs_of(oH, oW)
    wm = jnp.transpose(blk['conv1_w'].astype(jnp.bfloat16),
                       (2, 3, 4, 1, 0)).reshape(27 * Cin, Cout)
    wd = blk['down_w'].reshape(Cout, Cin).T.astype(jnp.bfloat16)
    sspec = pl.BlockSpec((1, Cout), lambda n, t: (0, 0))
    out1, res = pl.pallas_call(
        functools.partial(_conv_s2_kernel, oH=oH, oW=oW, Tpo=Tpo),
        out_shape=(jax.ShapeDtypeStruct((N, Tpo, RSo, Cout), jnp.bfloat16),
                   jax.ShapeDtypeStruct((N, Tpo, RSo, Cout), jnp.bfloat16)),
        grid=(N, Tpo),
        in_specs=[
            pl.BlockSpec((1, 1, 4, RQ, Cin),
                         lambda n, t: (n, jnp.clip(2 * t - 2, 0, Tpi - 1),
                                       0, 0, 0)),
            pl.BlockSpec((1, 1, 4, RQ, Cin),
                         lambda n, t: (n, jnp.clip(2 * t - 1, 0, Tpi - 1),
                                       0, 0, 0)),
            pl.BlockSpec((1, 1, 4, RQ, Cin),
                         lambda n, t: (n, jnp.clip(2 * t, 0, Tpi - 1),
                                       0, 0, 0)),
            pl.BlockSpec((27 * Cin, Cout), lambda n, t: (0, 0)),
            sspec, sspec,
            pl.BlockSpec((Cin, Cout), lambda n, t: (0, 0)),
            sspec, sspec,
        ],
        out_specs=(pl.BlockSpec((1, 1, RSo, Cout), lambda n, t: (n, t, 0, 0)),
                   pl.BlockSpec((1, 1, RSo, Cout),
                                lambda n, t: (n, t, 0, 0))),
        compiler_params=pltpu.CompilerParams(
            dimension_semantics=("parallel", "arbitrary"),
            vmem_limit_bytes=100 * 1024 * 1024),
    )(xq, xq, xq, wm,
      blk['bn1_scale'].reshape(1, Cout).astype(jnp.float32),
      blk['bn1_shift'].reshape(1, Cout).astype(jnp.float32),
      wd,
      blk['down_bn_scale'].reshape(1, Cout).astype(jnp.float32),
      blk['down_bn_shift'].reshape(1, Cout).astype(jnp.float32))
    return out1, res


def _stem_kernel(x0_ref, x1_ref, x2_ref, w_ref, s_ref, t_ref, o_ref, *, Tp):
    tp = pl.program_id(1)
    H = W = 56
    Wp = W + 2
    RS = o_ref.shape[2]
    M = RS - (2 * Wp + 2)
    D = Wp + 1

    @pl.when(jnp.logical_or(tp == 0, tp == Tp - 1))
    def _():
        o_ref[...] = jnp.zeros_like(o_ref)

    @pl.when(jnp.logical_and(tp > 0, tp < Tp - 1))
    def _():
        pieces = []
        for x_ref in (x0_ref, x1_ref, x2_ref):
            xv = x_ref[0, 0]
            for av in range(4):
                pieces.append(xv[av * Wp:av * Wp + M, :])
        col = jnp.concatenate(pieces, axis=-1)
        acc = jnp.dot(col, w_ref[...], preferred_element_type=jnp.float32)
        y = jnp.maximum(acc * s_ref[...] + t_ref[...], 0.0)
        y = jnp.where(_interior_mask(y.shape, H=H, W=W), y, 0.0)
        C = y.shape[1]
        o_ref[0, 0, 0:D, :] = jnp.zeros((D, C), o_ref.dtype)
        o_ref[0, 0, D:D + M, :] = y.astype(o_ref.dtype)
        o_ref[0, 0, D + M:RS, :] = jnp.zeros((RS - D - M, C), o_ref.dtype)


def _stem(x, stem_w, stem_scale, stem_shift):
    N = x.shape[0]
    RS1 = _rs_of(56, 56)
    M = RS1 - (2 * 58 + 2)
    xp = jnp.pad(x.astype(jnp.bfloat16),
                 ((0, 0), (0, 0), (1, 1), (3, 9), (3, 9)))
    xq = xp.reshape(N, 3, 10, 62, 2, 62, 2)
    xq = jnp.transpose(xq, (0, 2, 3, 5, 4, 6, 1))
    xq = jnp.pad(xq, ((0, 0),) * 6 + ((0, 1),))
    xq = xq.reshape(N, 10, 62, 62, 16)
    x4 = jnp.concatenate([xq[:, :, :, au:au + 58] for au in range(4)],
                         axis=-1)
    RQ = _rup(62 * 58, 16)
    x4 = jnp.pad(x4.reshape(N, 10, 62 * 58, 64),
                 ((0, 0), (0, 0), (0, RQ - 62 * 58), (0, 0)))
    wp = jnp.pad(stem_w, ((0, 0), (0, 1), (0, 0), (0, 1), (0, 1)))
    wp = wp.reshape(64, 4, 3, 4, 2, 4, 2)
    wp = jnp.transpose(wp, (2, 3, 5, 4, 6, 1, 0)).reshape(768, 64)
    Tp = 10
    return pl.pallas_call(
        functools.partial(_stem_kernel, Tp=Tp),
        out_shape=jax.ShapeDtypeStruct((N, Tp, RS1, 64), jnp.bfloat16),
        grid=(N, Tp),
        in_specs=[
            pl.BlockSpec((1, 1, RQ, 64),
                         lambda n, t: (n, jnp.maximum(t - 1, 0), 0, 0)),
            pl.BlockSpec((1, 1, RQ, 64), lambda n, t: (n, t, 0, 0)),
            pl.BlockSpec((1, 1, RQ, 64),
                         lambda n, t: (n, jnp.minimum(t + 1, Tp - 1), 0, 0)),
            pl.BlockSpec((768, 64), lambda n, t: (0, 0)),
            pl.BlockSpec((1, 64), lambda n, t: (0, 0)),
            pl.BlockSpec((1, 64), lambda n, t: (0, 0)),
        ],
        out_specs=pl.BlockSpec((1, 1, RS1, 64), lambda n, t: (n, t, 0, 0)),
        compiler_params=pltpu.CompilerParams(
            dimension_semantics=("parallel", "arbitrary"),
            vmem_limit_bytes=100 * 1024 * 1024),
    )(x4, x4, x4, wp.astype(jnp.bfloat16),
      stem_scale.reshape(1, 64).astype(jnp.float32),
      stem_shift.reshape(1, 64).astype(jnp.float32))


def _conv_pool_fc_kernel(x0_ref, x1_ref, x2_ref, w_ref, s_ref, t_ref,
                         res_ref, fcw_ref, fcb_ref, o_ref, *, H, W):
    Wp = W + 2
    RS = x1_ref.shape[2]
    M = RS - (2 * Wp + 2)
    D = Wp + 1
    acc = _col_and_acc((x0_ref, x1_ref, x2_ref), w_ref, M=M, Wp=Wp)
    y = acc * s_ref[...] + t_ref[...]
    y = y + res_ref[0, 0, D:D + M, :].astype(jnp.float32)
    y = jnp.maximum(y, 0.0)
    y = jnp.where(_interior_mask(y.shape, H=H, W=W), y, 0.0)
    pooled = jnp.sum(y, axis=0, keepdims=True) * (1.0 / (H * W))
    o_ref[0] = (jnp.dot(pooled, fcw_ref[...],
                        preferred_element_type=jnp.float32) + fcb_ref[...])


def _conv_pool_fc(x, w, scale, shift, residual, fc_w, fc_b, *, H, W):
    N, Tp, RS, Cin = x.shape
    Cout = w.shape[0]
    nc = fc_w.shape[1]
    NCp = _rup(nc, 128)
    wm = jnp.transpose(w.astype(jnp.bfloat16),
                       (2, 3, 4, 1, 0)).reshape(27 * Cin, Cout)
    fcw = jnp.pad(fc_w.astype(jnp.float32), ((0, 0), (0, NCp - nc)))
    fcb = jnp.pad(fc_b.astype(jnp.float32), (0, NCp - nc)).reshape(1, NCp)
    out = pl.pallas_call(
        functools.partial(_conv_pool_fc_kernel, H=H, W=W),
        out_shape=jax.ShapeDtypeStruct((N, 1, NCp), jnp.float32),
        grid=(N,),
        in_specs=[
            pl.BlockSpec((1, 1, RS, Cin), lambda n: (n, 0, 0, 0)),
            pl.BlockSpec((1, 1, RS, Cin), lambda n: (n, 1, 0, 0)),
            pl.BlockSpec((1, 1, RS, Cin), lambda n: (n, 2, 0, 0)),
            pl.BlockSpec((27 * Cin, Cout), lambda n: (0, 0)),
            pl.BlockSpec((1, Cout), lambda n: (0, 0)),
            pl.BlockSpec((1, Cout), lambda n: (0, 0)),
            pl.BlockSpec((1, 1, RS, Cout), lambda n: (n, 1, 0, 0)),
            pl.BlockSpec((Cout, NCp), lambda n: (0, 0)),
            pl.BlockSpec((1, NCp), lambda n: (0, 0)),
        ],
        out_specs=pl.BlockSpec((1, 1, NCp), lambda n: (n, 0, 0)),
        compiler_params=pltpu.CompilerParams(
            dimension_semantics=("parallel",),
            vmem_limit_bytes=100 * 1024 * 1024),
    )(x, x, x, wm, scale.reshape(1, Cout).astype(jnp.float32),
      shift.reshape(1, Cout).astype(jnp.float32), residual, fcw, fcb)
    return out[:, 0, :nc]


def _to_padded_flat(rows, N, T, H, W, C):
    Wp, Hp = W + 2, H + 2
    RS = _rs_of(H, W)
    x5 = rows.reshape(N, T, H, W, C)
    xp = jnp.pad(x5, ((0, 0), (1, 1), (1, 1), (1, 1), (0, 0)))
    flat = xp.reshape(N, T + 2, Hp * Wp, C)
    return jnp.pad(flat, ((0, 0), (0, 0), (0, RS - Hp * Wp), (0, 0)))


def _from_padded_flat(x_flat, N, T, H, W, C):
    Hp, Wp = H + 2, W + 2
    return x_flat[:, :, :Hp * Wp, :].reshape(N, T + 2, Hp, Wp, C)


def _strided_block0(x_flat, blk, dims_in, dims_out):
    _, oT, oH, oW = dims_out
    out1_flat, res_flat = _conv_s2(x_flat, blk, dims_in, dims_out)
    return _conv_s1(out1_flat, blk['conv2_w'], blk['bn2_scale'],
                    blk['bn2_shift'], H=oH, W=oW, residual=res_flat)


def kernel(x, stem_w, stem_scale, stem_shift,
           layer1_0_conv1_w, layer1_0_bn1_scale, layer1_0_bn1_shift,
           layer1_0_conv2_w, layer1_0_bn2_scale, layer1_0_bn2_shift,
           layer1_1_conv1_w, layer1_1_bn1_scale, layer1_1_bn1_shift,
           layer1_1_conv2_w, layer1_1_bn2_scale, layer1_1_bn2_shift,
           layer2_0_conv1_w, layer2_0_bn1_scale, layer2_0_bn1_shift,
           layer2_0_conv2_w, layer2_0_bn2_scale, layer2_0_bn2_shift,
           layer2_0_down_w, layer2_0_down_bn_scale, layer2_0_down_bn_shift,
           layer2_1_conv1_w, layer2_1_bn1_scale, layer2_1_bn1_shift,
           layer2_1_conv2_w, layer2_1_bn2_scale, layer2_1_bn2_shift,
           layer3_0_conv1_w, layer3_0_bn1_scale, layer3_0_bn1_shift,
           layer3_0_conv2_w, layer3_0_bn2_scale, layer3_0_bn2_shift,
           layer3_0_down_w, layer3_0_down_bn_scale, layer3_0_down_bn_shift,
           layer3_1_conv1_w, layer3_1_bn1_scale, layer3_1_bn1_shift,
           layer3_1_conv2_w, layer3_1_bn2_scale, layer3_1_bn2_shift,
           layer4_0_conv1_w, layer4_0_bn1_scale, layer4_0_bn1_shift,
           layer4_0_conv2_w, layer4_0_bn2_scale, layer4_0_bn2_shift,
           layer4_0_down_w, layer4_0_down_bn_scale, layer4_0_down_bn_shift,
           layer4_1_conv1_w, layer4_1_bn1_scale, layer4_1_bn1_shift,
           layer4_1_conv2_w, layer4_1_bn2_scale, layer4_1_bn2_shift,
           fc_w, fc_b):
    N = x.shape[0]
    h = _stem(x, stem_w, stem_scale, stem_shift)

    o1 = _conv_s1(h, layer1_0_conv1_w, layer1_0_bn1_scale, layer1_0_bn1_shift,
                  H=56, W=56)
    h = _conv_s1(o1, layer1_0_conv2_w, layer1_0_bn2_scale, layer1_0_bn2_shift,
                 H=56, W=56, residual=h)
    o1 = _conv_s1(h, layer1_1_conv1_w, layer1_1_bn1_scale, layer1_1_bn1_shift,
                  H=56, W=56)
    h = _conv_s1(o1, layer1_1_conv2_w, layer1_1_bn2_scale, layer1_1_bn2_shift,
                 H=56, W=56, residual=h)

    h = _strided_block0(
        h, {'conv1_w': layer2_0_conv1_w, 'bn1_scale': layer2_0_bn1_scale,
            'bn1_shift': layer2_0_bn1_shift, 'conv2_w': layer2_0_conv2_w,
            'bn2_scale': layer2_0_bn2_scale, 'bn2_shift': layer2_0_bn2_shift,
            'down_w': layer2_0_down_w, 'down_bn_scale': layer2_0_down_bn_scale,
            'down_bn_shift': layer2_0_down_bn_shift},
        (N, 8, 56, 56), (N, 4, 28, 28))
    o1 = _conv_s1(h, layer2_1_conv1_w, layer2_1_bn1_scale, layer2_1_bn1_shift,
                  H=28, W=28)
    h = _conv_s1(o1, layer2_1_conv2_w, layer2_1_bn2_scale, layer2_1_bn2_shift,
                 H=28, W=28, residual=h)

    h = _strided_block0(
        h, {'conv1_w': layer3_0_conv1_w, 'bn1_scale': layer3_0_bn1_scale,
            'bn1_shift': layer3_0_bn1_shift, 'conv2_w': layer3_0_conv2_w,
            'bn2_scale': layer3_0_bn2_scale, 'bn2_shift': layer3_0_bn2_shift,
            'down_w': layer3_0_down_w, 'down_bn_scale': layer3_0_down_bn_scale,
            'down_bn_shift': layer3_0_down_bn_shift},
        (N, 4, 28, 28), (N, 2, 14, 14))
    o1 = _conv_s1(h, layer3_1_conv1_w, layer3_1_bn1_scale, layer3_1_bn1_shift,
                  H=14, W=14)
    h = _conv_s1(o1, layer3_1_conv2_w, layer3_1_bn2_scale, layer3_1_bn2_shift,
                 H=14, W=14, residual=h)

    h = _strided_block0(
        h, {'conv1_w': layer4_0_conv1_w, 'bn1_scale': layer4_0_bn1_scale,
            'bn1_shift': layer4_0_bn1_shift, 'conv2_w': layer4_0_conv2_w,
            'bn2_scale': layer4_0_bn2_scale, 'bn2_shift': layer4_0_bn2_shift,
            'down_w': layer4_0_down_w, 'down_bn_scale': layer4_0_down_bn_scale,
            'down_bn_shift': layer4_0_down_bn_shift},
        (N, 2, 14, 14), (N, 1, 7, 7))
    o1 = _conv_s1(h, layer4_1_conv1_w, layer4_1_bn1_scale, layer4_1_bn1_shift,
                  H=7, W=7)
    logits = _conv_pool_fc(o1, layer4_1_conv2_w, layer4_1_bn2_scale,
                           layer4_1_bn2_shift, h, fc_w, fc_b, H=7, W=7)
    return logits
